# baseline (speedup 1.0000x reference)
import functools

import jax
import jax.numpy as jnp
from jax import lax
from jax.experimental import pallas as pl
from jax.experimental.pallas import tpu as pltpu
from jax.experimental.pallas import tpu_sc as plsc

NC, NS, L = 2, 16, 16
NW = NC * NS

N = 10000
E = 320000
H = 64
B = 512
AW = 80
EPT = E // NW
GC = 80
NCH = EPT // GC
RPT = N // NS
NP = 10240
RP3 = NP // NW

_mesh = plsc.VectorSubcoreMesh(core_axis_name="c", subcore_axis_name="s")


def _lk(v):
    return jnp.maximum(v, 0.01 * v)


def _zero_msg(msg_v, nrow, width):
    z = jnp.zeros((L,), jnp.float32)

    def zrow(i, _):
        for j in range(width // L):
            msg_v[i, pl.ds(j * L, L)] = z
        return 0

    lax.fori_loop(0, nrow, zrow, 0)


def _zero_acc_stripe(msg_v, acc_sh, sid):
    off = 0
    for nblk in (80, 80, 80, 80, 80, 80, 80, 65):
        pltpu.sync_copy(msg_v.at[pl.ds(0, nblk)],
                        acc_sh.at[pl.ds(sid * RPT + off, nblk)])
        off += nblk


NCHP = 128
NBUF = 2


@functools.partial(
    pl.kernel,
    out_type=jax.ShapeDtypeStruct((NC, NS, RPT, AW), jnp.float32),
    mesh=_mesh,
    compiler_params=pltpu.CompilerParams(needs_layout_passes=False,
                                         use_tc_tiling_on_sc=False),
    scratch_types=[
        pltpu.VMEM((EPT,), jnp.int32),
        pltpu.VMEM((NCHP, GC), jnp.int32),
        pltpu.VMEM((NBUF, GC, H), jnp.float32),
        pltpu.VMEM((NBUF, GC, H), jnp.float32),
        pltpu.VMEM((NBUF, GC, AW), jnp.float32),
        pltpu.VMEM((N,), jnp.float32),
        pltpu.VMEM((H,), jnp.float32),
        pltpu.VMEM_SHARED((N, AW), jnp.float32),
    ] + [pltpu.SemaphoreType.DMA] * (2 * NBUF),
)
def _gate_edge_sc(xa_hbm, ea_hbm, src_hbm, dst2_hbm, ad_hbm, attl_hbm,
                  out_hbm, src_v, dst_v, rows_v, ea_v, msg_v,
                  ad_v, attl_v, acc_sh, *sems):
    cid = lax.axis_index("c")
    sid = lax.axis_index("s")
    wid = sid * NC + cid
    base = wid * EPT

    _zero_msg(msg_v.at[0], GC, AW)
    _zero_acc_stripe(msg_v.at[0], acc_sh, sid)

    pltpu.sync_copy(src_hbm.at[pl.ds(base, EPT)], src_v)
    pltpu.sync_copy(dst2_hbm.at[wid], dst_v)
    pltpu.sync_copy(ad_hbm, ad_v)
    pltpu.sync_copy(attl_hbm, attl_v)
    plsc.subcore_barrier()

    iota = lax.iota(jnp.int32, L)
    sems_e = sems[:NBUF]
    sems_g = sems[NBUF:]

    def issue(ch, sl):
        pltpu.async_copy(ea_hbm.at[pl.ds(base + ch * GC, GC)], ea_v.at[sl],
                         sems_e[sl])
        pltpu.async_copy(xa_hbm.at[src_v.at[pl.ds(ch * GC, GC)]],
                         rows_v.at[sl], sems_g[sl])

    def wait_slot(sl):
        pltpu.make_async_copy(ea_hbm.at[pl.ds(base, GC)], ea_v.at[sl],
                              sems_e[sl]).wait()
        pltpu.make_async_copy(ea_hbm.at[pl.ds(base, GC)], rows_v.at[sl],
                              sems_g[sl]).wait()

    def work(ch, sl):
        def pa(e, _):
            for j in range(H // L):
                sj = pl.ds(j * L, L)
                v = rows_v[sl, e, sj] + ea_v[sl, e, sj]
                msg_v[sl, e, sj] = jnp.maximum(v, 0.01 * v)
            return 0

        lax.fori_loop(0, GC, pa, 0)

        msg2d = msg_v.at[sl]

        def pb(g, _):
            o = pl.multiple_of(g * L, L)
            e16 = o + iota
            d16 = dst_v[ch, pl.ds(o, L)]
            t = plsc.load_gather(ad_v, [d16])
            for k in range(H // L):
                al = attl_v[pl.ds(k * L, L)]
                for j in range(L):
                    col = jnp.full((L,), k * L + j, jnp.int32)
                    t = t + plsc.load_gather(msg2d, [e16, col]) * al[j]
            exv16 = jnp.exp(jnp.maximum(t, 0.01 * t))
            for lane in range(L):
                e = o + lane
                exv = jnp.full((L,), exv16[lane], jnp.float32)
                for j in range(H // L):
                    sj = pl.ds(j * L, L)
                    msg_v[sl, e, sj] = msg_v[sl, e, sj] * exv
                msg_v[sl, e, pl.ds(H, L)] = jnp.where(iota == 0, exv, 0.0)
            return 0

        lax.fori_loop(0, GC // L, pb, 0)
        pltpu.sync_copy(msg_v.at[sl], acc_sh.at[dst_v.at[ch]], add=True)

    for c0 in range(NBUF - 1):
        issue(c0, c0)

    def chunk_body(ch, _):
        for par in range(NBUF):
            @pl.when(lax.rem(ch, NBUF) == par)
            def _():
                @pl.when(ch + NBUF - 1 < NCH)
                def _():
                    issue(ch + NBUF - 1, (par + NBUF - 1) % NBUF)
                wait_slot(par)
                work(ch, par)
        return 0

    lax.fori_loop(0, NCH, chunk_body, 0)
    plsc.subcore_barrier()
    pltpu.sync_copy(acc_sh.at[pl.ds(sid * RPT, RPT)], out_hbm.at[cid, sid])


@functools.partial(
    pl.kernel,
    out_type=jax.ShapeDtypeStruct((NC, NS, RPT, AW), jnp.float32),
    mesh=_mesh,
    compiler_params=pltpu.CompilerParams(needs_layout_passes=False,
                                         use_tc_tiling_on_sc=False),
    scratch_types=[
        pltpu.VMEM((EPT,), jnp.int32),
        pltpu.VMEM((NCHP, GC), jnp.int32),
        pltpu.VMEM((NBUF, GC, H), jnp.float32),
        pltpu.VMEM((NBUF, GC, AW), jnp.float32),
        pltpu.VMEM((N,), jnp.float32),
        pltpu.VMEM((N,), jnp.float32),
        pltpu.VMEM_SHARED((N, AW), jnp.float32),
    ] + [pltpu.SemaphoreType.DMA] * NBUF,
)
def _att_edge_sc(x_hbm, src_hbm, dst2_hbm, as_hbm, ad_hbm, out_hbm,
                 src_v, dst_v, rows_v, msg_v, as_v, ad_v, acc_sh, *sems):
    cid = lax.axis_index("c")
    sid = lax.axis_index("s")
    wid = sid * NC + cid
    base = wid * EPT

    _zero_msg(msg_v.at[0], GC, AW)
    _zero_acc_stripe(msg_v.at[0], acc_sh, sid)

    pltpu.sync_copy(src_hbm.at[pl.ds(base, EPT)], src_v)
    pltpu.sync_copy(dst2_hbm.at[wid], dst_v)
    pltpu.sync_copy(as_hbm, as_v)
    pltpu.sync_copy(ad_hbm, ad_v)
    plsc.subcore_barrier()

    iota = lax.iota(jnp.int32, L)

    def issue(ch, sl):
        pltpu.async_copy(x_hbm.at[src_v.at[pl.ds(ch * GC, GC)]],
                         rows_v.at[sl], sems[sl])

    def wait_slot(sl):
        pltpu.make_async_copy(x_hbm.at[pl.ds(0, GC)], rows_v.at[sl],
                              sems[sl]).wait()

    def work(ch, sl):
        def grp(g, _):
            o = pl.multiple_of(g * L, L)
            s16 = src_v[pl.ds(pl.multiple_of(ch * GC + g * L, L), L)]
            d16 = dst_v[ch, pl.ds(o, L)]
            a = plsc.load_gather(as_v, [s16]) + plsc.load_gather(ad_v, [d16])
            exv16 = jnp.exp(jnp.maximum(a, 0.01 * a))
            for lane in range(L):
                e = o + lane
                exv = jnp.full((L,), exv16[lane], jnp.float32)
                for j in range(H // L):
                    sj = pl.ds(j * L, L)
                    msg_v[sl, e, sj] = rows_v[sl, e, sj] * exv
                msg_v[sl, e, pl.ds(H, L)] = jnp.where(iota == 0, exv, 0.0)
            return 0

        lax.fori_loop(0, GC // L, grp, 0)
        pltpu.sync_copy(msg_v.at[sl], acc_sh.at[dst_v.at[ch]], add=True)

    for c0 in range(NBUF - 1):
        issue(c0, c0)

    def chunk_body(ch, _):
        for par in range(NBUF):
            @pl.when(lax.rem(ch, NBUF) == par)
            def _():
                @pl.when(ch + NBUF - 1 < NCH)
                def _():
                    issue(ch + NBUF - 1, (par + NBUF - 1) % NBUF)
                wait_slot(par)
                work(ch, par)
        return 0

    lax.fori_loop(0, NCH, chunk_body, 0)
    plsc.subcore_barrier()
    pltpu.sync_copy(acc_sh.at[pl.ds(sid * RPT, RPT)], out_hbm.at[cid, sid])


@functools.partial(
    pl.kernel,
    out_type=jax.ShapeDtypeStruct((NW, B + 1, AW), jnp.float32),
    mesh=_mesh,
    compiler_params=pltpu.CompilerParams(needs_layout_passes=False, use_tc_tiling_on_sc=False),
    scratch_types=[
        pltpu.VMEM((RP3, H), jnp.float32),
        pltpu.VMEM((RP3,), jnp.int32),
        pltpu.VMEM((RP3,), jnp.float32),
        pltpu.VMEM((B + 16,), jnp.float32),
        pltpu.VMEM((B + 1, AW), jnp.float32),
    ],
)
def _pool_att_sc(x_hbm, b_hbm, as_hbm, adt_hbm, out_hbm,
                 rows_v, b_v, as_v, adt_v, acc_v):
    cid = lax.axis_index("c")
    sid = lax.axis_index("s")
    wid = sid * NC + cid
    base = wid * RP3

    z = jnp.zeros((L,), jnp.float32)

    def zrow(i, _):
        for j in range(AW // L):
            acc_v[i, pl.ds(j * L, L)] = z
        return 0

    lax.fori_loop(0, B + 1, zrow, 0)

    pltpu.sync_copy(x_hbm.at[pl.ds(base, RP3)], rows_v)
    pltpu.sync_copy(b_hbm.at[pl.ds(base, RP3)], b_v)
    pltpu.sync_copy(as_hbm.at[pl.ds(base, RP3)], as_v)
    pltpu.sync_copy(adt_hbm, adt_v)

    iota = lax.iota(jnp.int32, L)

    def pg(g, _):
        o = pl.multiple_of(g * L, L)
        b16 = b_v[pl.ds(o, L)]
        a = as_v[pl.ds(o, L)] + plsc.load_gather(adt_v, [b16])
        exv16 = jnp.exp(jnp.maximum(a, 0.01 * a))
        for lane in range(L):
            e = o + lane
            de = b16[lane]
            exv = jnp.full((L,), exv16[lane], jnp.float32)
            for j in range(H // L):
                sl = pl.ds(j * L, L)
                acc_v[de, sl] = acc_v[de, sl] + rows_v[e, sl] * exv
            sl = pl.ds(H, L)
            acc_v[de, sl] = acc_v[de, sl] + jnp.where(iota == 0, exv, 0.0)
        return 0

    lax.fori_loop(0, RP3 // L, pg, 0)

    pltpu.sync_copy(acc_v, out_hbm.at[wid])


def _gru(xv, h, Wih, Whh, bih, bhh):
    gi = xv @ Wih.T + bih
    gh = h @ Whh.T + bhh
    ir, iz, inn = jnp.split(gi, 3, axis=1)
    hr, hz, hn = jnp.split(gh, 3, axis=1)
    r = jax.nn.sigmoid(ir + hr)
    zz = jax.nn.sigmoid(iz + hz)
    n_ = jnp.tanh(inn + r * hn)
    return (1.0 - zz) * n_ + zz * h


def kernel(x, edge_index, edge_attr, batch, gene, taxonomy, duration,
           W1, b1, ge_W1, ge_W2, ge_att_l, ge_att_r, ge_bias,
           gru_Wih, gru_Whh, gru_bih, gru_bhh,
           ac_W, ac_att_src, ac_att_dst, ac_bias,
           ag_Wih, ag_Whh, ag_bih, ag_bhh,
           mc_W, mc_att_src, mc_att_dst, mc_bias,
           mg_Wih, mg_Whh, mg_bih, mg_bhh,
           gc_W, gc_b, W_dur, b_dur, W4, b4, W5, b5):
    f32 = jnp.float32
    src = edge_index[0]
    dst = edge_index[1]
    dst2 = jnp.pad(dst.reshape(NW, NCH, GC), ((0, 0), (0, NCHP - NCH), (0, 0)))

    W1a = ge_W1[:, :H]
    W1b = ge_W1[:, H:]
    x1 = _lk(x @ W1.T + b1)
    xa = x1 @ W1a.T
    ad_g = x1 @ ge_att_r
    ea = edge_attr @ W1b.T

    acc = _gate_edge_sc(xa, ea, src, dst2, ad_g, ge_att_l)
    acc = acc.reshape(NC, N, AW)
    a0 = acc[0] + acc[1]
    u = a0[:, :H] / (a0[:, H:H + 1] + 1e-16)
    h1 = jax.nn.elu(u @ ge_W2.T + ge_bias)
    x2 = jax.nn.relu(_gru(h1, x1, gru_Wih, gru_Whh, gru_bih, gru_bhh))

    as2 = x2 @ (ac_W.T @ ac_att_src)
    ad2 = x2 @ (ac_W.T @ ac_att_dst)
    acc2 = _att_edge_sc(x2, src, dst2, as2, ad2).reshape(NC, N, AW)
    a1 = acc2[0] + acc2[1]
    h2 = jax.nn.elu((a1[:, :H] / (a1[:, H:H + 1] + 1e-16)) @ ac_W.T + ac_bias)
    x3 = jax.nn.relu(_gru(h2, x2, ag_Wih, ag_Whh, ag_bih, ag_bhh))

    x3p = jnp.concatenate([x3, jnp.zeros((NP - N, H), f32)], 0)
    bp = jnp.concatenate([batch, jnp.full((NP - N,), B, jnp.int32)], 0)
    zs = jnp.zeros((NP,), f32)
    zt = jnp.zeros((B + 16,), f32)

    p = _pool_att_sc(x3p, bp, zs, zt).sum(0)
    out_g = jax.nn.relu(p[:B, :H])

    as3 = x3 @ (mc_W.T @ mc_att_src)
    as3p = jnp.concatenate([as3, jnp.zeros((NP - N,), f32)], 0)
    wdst = mc_W.T @ mc_att_dst
    for _ in range(2):
        adg = jnp.pad(out_g @ wdst, (0, 16))
        m = _pool_att_sc(x3p, bp, as3p, adg).sum(0)
        h = jax.nn.elu((m[:B, :H] / (m[:B, H:H + 1] + 1e-16)) @ mc_W.T
                       + mc_bias)
        out_g = jax.nn.relu(_gru(h, out_g, mg_Wih, mg_Whh, mg_bih, mg_bhh))

    g = gene[:, :, :3072].reshape(B, 4, 1024, 3)
    g = jnp.einsum('bclk,ck->bl', g, gc_W) + gc_b[0]
    gp = g.reshape(B, H, 16).mean(-1)
    dur = jax.nn.relu(duration @ W_dur.T + b_dur)
    cat = jnp.concatenate([out_g, gp, taxonomy, dur], 1)
    return (cat @ W4.T + b4) @ W5.T + b5


# speedup vs baseline: 1.0727x; 1.0727x over previous
import functools

import jax
import jax.numpy as jnp
from jax import lax
from jax.experimental import pallas as pl
from jax.experimental.pallas import tpu as pltpu
from jax.experimental.pallas import tpu_sc as plsc

NC, NS, L = 2, 16, 16
NW = NC * NS

N = 10000
E = 320000
H = 64
B = 512
AW = 80
EPT = E // NW
GC = 80
NCH = EPT // GC
RPT = N // NS
NP = 10240
DUR = 8
RP3 = NP // NW

_mesh = plsc.VectorSubcoreMesh(core_axis_name="c", subcore_axis_name="s")


def _lk(v):
    return jnp.maximum(v, 0.01 * v)


def _zero_msg(msg_v, nrow, width):
    z = jnp.zeros((L,), jnp.float32)

    def zrow(i, _):
        for j in range(width // L):
            msg_v[i, pl.ds(j * L, L)] = z
        return 0

    lax.fori_loop(0, nrow, zrow, 0)


def _zero_acc_stripe(msg_v, acc_sh, sid):
    off = 0
    for nblk in (80, 80, 80, 80, 80, 80, 80, 65):
        pltpu.sync_copy(msg_v.at[pl.ds(0, nblk)],
                        acc_sh.at[pl.ds(sid * RPT + off, nblk)])
        off += nblk


NCHP = 128
NBUF = 2


@functools.partial(
    pl.kernel,
    out_type=jax.ShapeDtypeStruct((NC, NS, RPT, AW), jnp.float32),
    mesh=_mesh,
    compiler_params=pltpu.CompilerParams(needs_layout_passes=False,
                                         use_tc_tiling_on_sc=False),
    scratch_types=[
        pltpu.VMEM((EPT,), jnp.int32),
        pltpu.VMEM((NCHP, GC), jnp.int32),
        pltpu.VMEM((NBUF, GC, H), jnp.float32),
        pltpu.VMEM((NBUF, GC, H), jnp.float32),
        pltpu.VMEM((NBUF, GC, AW), jnp.float32),
        pltpu.VMEM((N,), jnp.float32),
        pltpu.VMEM((H,), jnp.float32),
        pltpu.VMEM_SHARED((N, AW), jnp.float32),
    ] + [pltpu.SemaphoreType.DMA] * (2 * NBUF),
)
def _gate_edge_sc(xa_hbm, ea_hbm, src_hbm, dst2_hbm, ad_hbm, attl_hbm,
                  out_hbm, src_v, dst_v, rows_v, ea_v, msg_v,
                  ad_v, attl_v, acc_sh, *sems):
    cid = lax.axis_index("c")
    sid = lax.axis_index("s")
    wid = sid * NC + cid
    base = wid * EPT

    _zero_msg(msg_v.at[0], GC, AW)
    _zero_acc_stripe(msg_v.at[0], acc_sh, sid)

    pltpu.sync_copy(src_hbm.at[pl.ds(base, EPT)], src_v)
    pltpu.sync_copy(dst2_hbm.at[wid], dst_v)
    pltpu.sync_copy(ad_hbm, ad_v)
    pltpu.sync_copy(attl_hbm, attl_v)
    plsc.subcore_barrier()

    iota = lax.iota(jnp.int32, L)
    sems_e = sems[:NBUF]
    sems_g = sems[NBUF:]

    def issue(ch, sl):
        pltpu.async_copy(ea_hbm.at[pl.ds(base + ch * GC, GC)], ea_v.at[sl],
                         sems_e[sl])
        pltpu.async_copy(xa_hbm.at[src_v.at[pl.ds(ch * GC, GC)]],
                         rows_v.at[sl], sems_g[sl])

    def wait_slot(sl):
        pltpu.make_async_copy(ea_hbm.at[pl.ds(base, GC)], ea_v.at[sl],
                              sems_e[sl]).wait()
        pltpu.make_async_copy(ea_hbm.at[pl.ds(base, GC)], rows_v.at[sl],
                              sems_g[sl]).wait()

    def work(ch, sl):
        def grp(g, _):
            o = pl.multiple_of(g * L, L)
            d16 = dst_v[ch, pl.ds(o, L)]
            adv = plsc.load_gather(ad_v, [d16])
            for lane in range(L):
                e = o + lane
                tv = jnp.zeros((L,), jnp.float32)
                hjs = []
                for j in range(H // L):
                    sj = pl.ds(j * L, L)
                    v = rows_v[sl, e, sj] + ea_v[sl, e, sj]
                    hj = jnp.maximum(v, 0.01 * v)
                    hjs.append(hj)
                    tv = tv + hj * attl_v[sj]
                t = jnp.sum(tv) + adv[lane]
                av = jnp.full((L,), t, jnp.float32)
                exv = jnp.exp(jnp.maximum(av, 0.01 * av))
                for j in range(H // L):
                    msg_v[sl, e, pl.ds(j * L, L)] = hjs[j] * exv
                msg_v[sl, e, pl.ds(H, L)] = jnp.where(iota == 0, exv, 0.0)
            return 0

        lax.fori_loop(0, GC // L, grp, 0)
        pltpu.sync_copy(msg_v.at[sl], acc_sh.at[dst_v.at[ch]], add=True)

    for c0 in range(NBUF - 1):
        issue(c0, c0)

    def chunk_body(ch, _):
        for par in range(NBUF):
            @pl.when(lax.rem(ch, NBUF) == par)
            def _():
                @pl.when(ch + NBUF - 1 < NCH)
                def _():
                    issue(ch + NBUF - 1, (par + NBUF - 1) % NBUF)
                wait_slot(par)
                work(ch, par)
        return 0

    lax.fori_loop(0, NCH, chunk_body, 0)
    plsc.subcore_barrier()
    pltpu.sync_copy(acc_sh.at[pl.ds(sid * RPT, RPT)], out_hbm.at[cid, sid])


@functools.partial(
    pl.kernel,
    out_type=jax.ShapeDtypeStruct((NC, NS, RPT, AW), jnp.float32),
    mesh=_mesh,
    compiler_params=pltpu.CompilerParams(needs_layout_passes=False,
                                         use_tc_tiling_on_sc=False),
    scratch_types=[
        pltpu.VMEM((EPT,), jnp.int32),
        pltpu.VMEM((NCHP, GC), jnp.int32),
        pltpu.VMEM((NBUF, GC, H), jnp.float32),
        pltpu.VMEM((NBUF, GC, AW), jnp.float32),
        pltpu.VMEM((N,), jnp.float32),
        pltpu.VMEM((N,), jnp.float32),
        pltpu.VMEM_SHARED((N, AW), jnp.float32),
    ] + [pltpu.SemaphoreType.DMA] * NBUF,
)
def _att_edge_sc(x_hbm, src_hbm, dst2_hbm, as_hbm, ad_hbm, out_hbm,
                 src_v, dst_v, rows_v, msg_v, as_v, ad_v, acc_sh, *sems):
    cid = lax.axis_index("c")
    sid = lax.axis_index("s")
    wid = sid * NC + cid
    base = wid * EPT

    _zero_msg(msg_v.at[0], GC, AW)
    _zero_acc_stripe(msg_v.at[0], acc_sh, sid)

    pltpu.sync_copy(src_hbm.at[pl.ds(base, EPT)], src_v)
    pltpu.sync_copy(dst2_hbm.at[wid], dst_v)
    pltpu.sync_copy(as_hbm, as_v)
    pltpu.sync_copy(ad_hbm, ad_v)
    plsc.subcore_barrier()

    iota = lax.iota(jnp.int32, L)

    def issue(ch, sl):
        pltpu.async_copy(x_hbm.at[src_v.at[pl.ds(ch * GC, GC)]],
                         rows_v.at[sl], sems[sl])

    def wait_slot(sl):
        pltpu.make_async_copy(x_hbm.at[pl.ds(0, GC)], rows_v.at[sl],
                              sems[sl]).wait()

    def work(ch, sl):
        def grp(g, _):
            o = pl.multiple_of(g * L, L)
            s16 = src_v[pl.ds(pl.multiple_of(ch * GC + g * L, L), L)]
            d16 = dst_v[ch, pl.ds(o, L)]
            a = plsc.load_gather(as_v, [s16]) + plsc.load_gather(ad_v, [d16])
            exv16 = jnp.exp(jnp.maximum(a, 0.01 * a))
            for lane in range(L):
                e = o + lane
                exv = jnp.full((L,), exv16[lane], jnp.float32)
                for j in range(H // L):
                    sj = pl.ds(j * L, L)
                    msg_v[sl, e, sj] = rows_v[sl, e, sj] * exv
                msg_v[sl, e, pl.ds(H, L)] = jnp.where(iota == 0, exv, 0.0)
            return 0

        lax.fori_loop(0, GC // L, grp, 0)
        pltpu.sync_copy(msg_v.at[sl], acc_sh.at[dst_v.at[ch]], add=True)

    for c0 in range(NBUF - 1):
        issue(c0, c0)

    def chunk_body(ch, _):
        for par in range(NBUF):
            @pl.when(lax.rem(ch, NBUF) == par)
            def _():
                @pl.when(ch + NBUF - 1 < NCH)
                def _():
                    issue(ch + NBUF - 1, (par + NBUF - 1) % NBUF)
                wait_slot(par)
                work(ch, par)
        return 0

    lax.fori_loop(0, NCH, chunk_body, 0)
    plsc.subcore_barrier()
    pltpu.sync_copy(acc_sh.at[pl.ds(sid * RPT, RPT)], out_hbm.at[cid, sid])


@functools.partial(
    pl.kernel,
    out_type=jax.ShapeDtypeStruct((NW, B + 1, AW), jnp.float32),
    mesh=_mesh,
    compiler_params=pltpu.CompilerParams(needs_layout_passes=False, use_tc_tiling_on_sc=False),
    scratch_types=[
        pltpu.VMEM((RP3, H), jnp.float32),
        pltpu.VMEM((RP3,), jnp.int32),
        pltpu.VMEM((RP3,), jnp.float32),
        pltpu.VMEM((B + 16,), jnp.float32),
        pltpu.VMEM((B + 1, AW), jnp.float32),
    ],
)
def _pool_att_sc(x_hbm, b_hbm, as_hbm, adt_hbm, out_hbm,
                 rows_v, b_v, as_v, adt_v, acc_v):
    cid = lax.axis_index("c")
    sid = lax.axis_index("s")
    wid = sid * NC + cid
    base = wid * RP3

    z = jnp.zeros((L,), jnp.float32)

    def zrow(i, _):
        for j in range(AW // L):
            acc_v[i, pl.ds(j * L, L)] = z
        return 0

    lax.fori_loop(0, B + 1, zrow, 0)

    pltpu.sync_copy(x_hbm.at[pl.ds(base, RP3)], rows_v)
    pltpu.sync_copy(b_hbm.at[pl.ds(base, RP3)], b_v)
    pltpu.sync_copy(as_hbm.at[pl.ds(base, RP3)], as_v)
    pltpu.sync_copy(adt_hbm, adt_v)

    iota = lax.iota(jnp.int32, L)

    def pg(g, _):
        o = pl.multiple_of(g * L, L)
        b16 = b_v[pl.ds(o, L)]
        a = as_v[pl.ds(o, L)] + plsc.load_gather(adt_v, [b16])
        exv16 = jnp.exp(jnp.maximum(a, 0.01 * a))
        for lane in range(L):
            e = o + lane
            de = b16[lane]
            exv = jnp.full((L,), exv16[lane], jnp.float32)
            for j in range(H // L):
                sl = pl.ds(j * L, L)
                acc_v[de, sl] = acc_v[de, sl] + rows_v[e, sl] * exv
            sl = pl.ds(H, L)
            acc_v[de, sl] = acc_v[de, sl] + jnp.where(iota == 0, exv, 0.0)
        return 0

    lax.fori_loop(0, RP3 // L, pg, 0)

    pltpu.sync_copy(acc_v, out_hbm.at[wid])


NB = 1000
NGRID = N // NB


def _full(spec_shape):
    nd = len(spec_shape)
    return pl.BlockSpec(spec_shape, lambda *_: (0,) * nd)


def _gru_block(xv, h, Wih, Whh, bih, bhh):
    gi = jnp.dot(xv, Wih.T, preferred_element_type=jnp.float32) + bih
    gh = jnp.dot(h, Whh.T, preferred_element_type=jnp.float32) + bhh
    r = jax.nn.sigmoid(gi[:, :H] + gh[:, :H])
    z = jax.nn.sigmoid(gi[:, H:2 * H] + gh[:, H:2 * H])
    n_ = jnp.tanh(gi[:, 2 * H:] + r * gh[:, 2 * H:])
    return (1.0 - z) * n_ + z * h


def _elu(v):
    return jnp.where(v > 0, v, jnp.exp(jnp.minimum(v, 0.0)) - 1.0)


def _tk_node_prep(x, W1, b1, W1a, attr):
    def body(x_r, W1_r, b1_r, W1a_r, attr_r, x1_r, xa_r, adg_r):
        x1 = _lk(jnp.dot(x_r[...], W1_r[...].T,
                         preferred_element_type=jnp.float32) + b1_r[...])
        x1_r[...] = x1
        xa_r[...] = jnp.dot(x1, W1a_r[...].T,
                            preferred_element_type=jnp.float32)
        adg_r[...] = jnp.sum(x1 * attr_r[...].T, axis=1, keepdims=True)

    return pl.pallas_call(
        body,
        grid=(NGRID,),
        in_specs=[pl.BlockSpec((NB, 128), lambda i: (i, 0)),
                  _full((H, 128)), _full((H,)), _full((H, H)),
                  _full((H, 1))],
        out_specs=[pl.BlockSpec((NB, H), lambda i: (i, 0)),
                   pl.BlockSpec((NB, H), lambda i: (i, 0)),
                   pl.BlockSpec((NB, 1), lambda i: (i, 0))],
        out_shape=[jax.ShapeDtypeStruct((N, H), jnp.float32),
                   jax.ShapeDtypeStruct((N, H), jnp.float32),
                   jax.ShapeDtypeStruct((N, 1), jnp.float32)],
    )(x, W1, b1, W1a, attr)


EB = 4000


def _tk_ea(edge_attr, W1b):
    def body(e_r, w_r, o_r):
        o_r[...] = jnp.dot(e_r[...], w_r[...].T,
                           preferred_element_type=jnp.float32)

    return pl.pallas_call(
        body,
        grid=(E // EB,),
        in_specs=[pl.BlockSpec((EB, 16), lambda i: (i, 0)),
                  _full((H, 16))],
        out_specs=pl.BlockSpec((EB, H), lambda i: (i, 0)),
        out_shape=jax.ShapeDtypeStruct((E, H), jnp.float32),
    )(edge_attr, W1b)


def _tk_conv_post(acc, xprev, Wc, bc, Wih, Whh, bih, bhh, Wn, asv, adv):
    def body(a0_r, a1_r, xp_r, Wc_r, bc_r, Wih_r, Whh_r, bih_r, bhh_r,
             Wn_r, as0_r, ad0_r, xn_r, as_r, ad_r):
        a = a0_r[0] + a1_r[0]
        u = a[:, :H] / (a[:, H:H + 1] + 1e-16)
        h = _elu(jnp.dot(u, Wc_r[...].T,
                         preferred_element_type=jnp.float32) + bc_r[...])
        xn = jnp.maximum(
            _gru_block(h, xp_r[...], Wih_r[...], Whh_r[...], bih_r[...],
                       bhh_r[...]), 0.0)
        xn_r[...] = xn
        ws = jnp.dot(Wn_r[...].T, as0_r[...],
                     preferred_element_type=jnp.float32)
        wd = jnp.dot(Wn_r[...].T, ad0_r[...],
                     preferred_element_type=jnp.float32)
        as_r[...] = jnp.sum(xn * ws.T, axis=1, keepdims=True)
        ad_r[...] = jnp.sum(xn * wd.T, axis=1, keepdims=True)

    return pl.pallas_call(
        body,
        grid=(NGRID,),
        in_specs=[pl.BlockSpec((1, NB, AW), lambda i: (0, i, 0)),
                  pl.BlockSpec((1, NB, AW), lambda i: (1, i, 0)),
                  pl.BlockSpec((NB, H), lambda i: (i, 0)),
                  _full((H, H)), _full((H,)),
                  _full((3 * H, H)), _full((3 * H, H)),
                  _full((3 * H,)), _full((3 * H,)),
                  _full((H, H)), _full((H, 1)), _full((H, 1))],
        out_specs=[pl.BlockSpec((NB, H), lambda i: (i, 0)),
                   pl.BlockSpec((NB, 1), lambda i: (i, 0)),
                   pl.BlockSpec((NB, 1), lambda i: (i, 0))],
        out_shape=[jax.ShapeDtypeStruct((N, H), jnp.float32),
                   jax.ShapeDtypeStruct((N, 1), jnp.float32),
                   jax.ShapeDtypeStruct((N, 1), jnp.float32)],
    )(acc, acc, xprev, Wc, bc, Wih, Whh, bih, bhh, Wn, asv, adv)


def _tk_pool_post(p32, Wmc, attd):
    def body(p_r, W_r, ad_r, og_r, adg_r):
        seg = jnp.sum(p_r[...], axis=0)[:B, :H]
        og = jnp.maximum(seg, 0.0)
        og_r[...] = og
        wv = jnp.dot(W_r[...].T, ad_r[...],
                     preferred_element_type=jnp.float32)
        adg_r[...] = jnp.sum(og * wv.T, axis=1, keepdims=True)

    return pl.pallas_call(
        body,
        in_specs=[_full((NW, B + 1, AW)), _full((H, H)), _full((H, 1))],
        out_specs=[_full((B, H)), _full((B, 1))],
        out_shape=[jax.ShapeDtypeStruct((B, H), jnp.float32),
                   jax.ShapeDtypeStruct((B, 1), jnp.float32)],
    )(p32, Wmc, attd)


def _tk_mc_post(m32, og, Wmc, bmc, Wih, Whh, bih, bhh, attd):
    def body(m_r, og_r, W_r, b_r, Wih_r, Whh_r, bih_r, bhh_r, ad_r,
             on_r, adg_r):
        a = jnp.sum(m_r[...], axis=0)[:B]
        u = a[:, :H] / (a[:, H:H + 1] + 1e-16)
        h = _elu(jnp.dot(u, W_r[...].T,
                         preferred_element_type=jnp.float32) + b_r[...])
        on = jnp.maximum(
            _gru_block(h, og_r[...], Wih_r[...], Whh_r[...], bih_r[...],
                       bhh_r[...]), 0.0)
        on_r[...] = on
        wv = jnp.dot(W_r[...].T, ad_r[...],
                     preferred_element_type=jnp.float32)
        adg_r[...] = jnp.sum(on * wv.T, axis=1, keepdims=True)

    return pl.pallas_call(
        body,
        in_specs=[_full((NW, B + 1, AW)), _full((B, H)),
                  _full((H, H)), _full((H,)),
                  _full((3 * H, H)), _full((3 * H, H)),
                  _full((3 * H,)), _full((3 * H,)), _full((H, 1))],
        out_specs=[_full((B, H)), _full((B, 1))],
        out_shape=[jax.ShapeDtypeStruct((B, H), jnp.float32),
                   jax.ShapeDtypeStruct((B, 1), jnp.float32)],
    )(m32, og, Wmc, bmc, Wih, Whh, bih, bhh, attd)


GB = 64
GL = 3072


def _tk_gene(gene, gc_W, gc_b):
    def body(g_r, w_r, b_r, o_r):
        m_idx = lax.broadcasted_iota(jnp.int32, (GL, H), 0)
        i_idx = lax.broadcasted_iota(jnp.int32, (GL, H), 1)
        mask = (m_idx // (GL // H) == i_idx).astype(jnp.float32) / 16.0
        rem = m_idx % 3
        acc = jnp.zeros((GB, H), jnp.float32)
        w = w_r[...]
        for c in range(4):
            vals = jnp.where(rem == 0, w[c, 0],
                             jnp.where(rem == 1, w[c, 1], w[c, 2]))
            acc = acc + jnp.dot(g_r[0, :, c, :], vals * mask,
                                preferred_element_type=jnp.float32)
        o_r[...] = acc + b_r[0]

    return pl.pallas_call(
        body,
        grid=(B // GB,),
        in_specs=[pl.BlockSpec((1, GB, 4, GL), lambda i: (0, i, 0, 0)),
                  _full((4, 3)), _full((1,))],
        out_specs=pl.BlockSpec((GB, H), lambda i: (i, 0)),
        out_shape=jax.ShapeDtypeStruct((B, H), jnp.float32),
    )(gene[None], gc_W, gc_b)


def _tk_final(og, gp, taxonomy, duration, W_dur, b_dur, W4, b4, W5, b5):
    def body(og_r, gp_r, tx_r, du_r, Wd_r, bd_r, W4_r, b4_r, W5_r, b5_r,
             o_r):
        dur = jnp.maximum(
            jnp.dot(du_r[...], Wd_r[...].T,
                    preferred_element_type=jnp.float32) + bd_r[...], 0.0)
        cat = jnp.concatenate([og_r[...], gp_r[...], tx_r[...], dur], 1)
        c4 = jnp.dot(cat, W4_r[...].T,
                     preferred_element_type=jnp.float32) + b4_r[...]
        o_r[...] = jnp.sum(c4 * W5_r[...], axis=1, keepdims=True) + b5_r[0]

    return pl.pallas_call(
        body,
        in_specs=[_full((B, H)), _full((B, H)), _full((B, H)),
                  _full((B, DUR)), _full((H, DUR)), _full((H,)),
                  _full((H, 4 * H)), _full((H,)),
                  _full((1, H)), _full((1,))],
        out_specs=_full((B, 1)),
        out_shape=jax.ShapeDtypeStruct((B, 1), jnp.float32),
    )(og, gp, taxonomy, duration, W_dur, b_dur, W4, b4, W5, b5)


def _gru(xv, h, Wih, Whh, bih, bhh):
    gi = xv @ Wih.T + bih
    gh = h @ Whh.T + bhh
    ir, iz, inn = jnp.split(gi, 3, axis=1)
    hr, hz, hn = jnp.split(gh, 3, axis=1)
    r = jax.nn.sigmoid(ir + hr)
    zz = jax.nn.sigmoid(iz + hz)
    n_ = jnp.tanh(inn + r * hn)
    return (1.0 - zz) * n_ + zz * h


def kernel(x, edge_index, edge_attr, batch, gene, taxonomy, duration,
           W1, b1, ge_W1, ge_W2, ge_att_l, ge_att_r, ge_bias,
           gru_Wih, gru_Whh, gru_bih, gru_bhh,
           ac_W, ac_att_src, ac_att_dst, ac_bias,
           ag_Wih, ag_Whh, ag_bih, ag_bhh,
           mc_W, mc_att_src, mc_att_dst, mc_bias,
           mg_Wih, mg_Whh, mg_bih, mg_bhh,
           gc_W, gc_b, W_dur, b_dur, W4, b4, W5, b5):
    f32 = jnp.float32
    src = edge_index[0]
    dst = edge_index[1]
    dst2 = jnp.pad(dst.reshape(NW, NCH, GC), ((0, 0), (0, NCHP - NCH), (0, 0)))

    W1a = ge_W1[:, :H]
    W1b = ge_W1[:, H:]
    x1, xa, adg = _tk_node_prep(x, W1, b1, W1a, ge_att_r.reshape(H, 1))
    ea = _tk_ea(edge_attr, W1b)

    acc = _gate_edge_sc(xa, ea, src, dst2, adg.reshape(N), ge_att_l)
    acc = acc.reshape(NC, N, AW)
    x2, as2, ad2 = _tk_conv_post(acc, x1, ge_W2, ge_bias,
                                 gru_Wih, gru_Whh, gru_bih, gru_bhh,
                                 ac_W, ac_att_src.reshape(H, 1),
                                 ac_att_dst.reshape(H, 1))

    acc2 = _att_edge_sc(x2, src, dst2, as2.reshape(N),
                        ad2.reshape(N)).reshape(NC, N, AW)
    x3, as3, _ = _tk_conv_post(acc2, x2, ac_W, ac_bias,
                               ag_Wih, ag_Whh, ag_bih, ag_bhh,
                               mc_W, mc_att_src.reshape(H, 1),
                               mc_att_src.reshape(H, 1))

    x3p = jnp.concatenate([x3, jnp.zeros((NP - N, H), f32)], 0)
    bp = jnp.concatenate([batch, jnp.full((NP - N,), B, jnp.int32)], 0)
    as3p = jnp.concatenate([as3.reshape(N), jnp.zeros((NP - N,), f32)], 0)
    zs = jnp.zeros((NP,), f32)
    zt = jnp.zeros((B + 16,), f32)

    p32 = _pool_att_sc(x3p, bp, zs, zt)
    out_g, adg1 = _tk_pool_post(p32, mc_W, mc_att_dst.reshape(H, 1))

    adgp = adg1
    for _ in range(2):
        m32 = _pool_att_sc(x3p, bp, as3p,
                           jnp.pad(adgp.reshape(B), (0, 16)))
        out_g, adgp = _tk_mc_post(m32, out_g, mc_W, mc_bias,
                                  mg_Wih, mg_Whh, mg_bih, mg_bhh,
                                  mc_att_dst.reshape(H, 1))

    gp = _tk_gene(gene, gc_W, gc_b)
    return _tk_final(out_g, gp, taxonomy, duration, W_dur, b_dur,
                     W4, b4, W5, b5)


# speedup vs baseline: 1.1319x; 1.0552x over previous
import functools

import jax
import jax.numpy as jnp
from jax import lax
from jax.experimental import pallas as pl
from jax.experimental.pallas import tpu as pltpu
from jax.experimental.pallas import tpu_sc as plsc

NC, NS, L = 2, 16, 16
NW = NC * NS

N = 10000
E = 320000
H = 64
B = 512
AW = 80
EPT = E // NW
GC = 80
NCH = EPT // GC
RPT = N // NS
NP = 10240
DUR = 8
RP3 = NP // NW

_mesh = plsc.VectorSubcoreMesh(core_axis_name="c", subcore_axis_name="s")


def _lk(v):
    return jnp.maximum(v, 0.01 * v)


def _zero_msg(msg_v, nrow, width):
    z = jnp.zeros((L,), jnp.float32)

    def zrow(i, _):
        for j in range(width // L):
            msg_v[i, pl.ds(j * L, L)] = z
        return 0

    lax.fori_loop(0, nrow, zrow, 0)


def _zero_acc_stripe(msg_v, acc_sh, sid):
    off = 0
    for nblk in (80, 80, 80, 80, 80, 80, 80, 65):
        pltpu.sync_copy(msg_v.at[pl.ds(0, nblk)],
                        acc_sh.at[pl.ds(sid * RPT + off, nblk)])
        off += nblk


NCHP = 128
NBUF = 2


@functools.partial(
    pl.kernel,
    out_type=jax.ShapeDtypeStruct((NC, NS, RPT, AW), jnp.float32),
    mesh=_mesh,
    compiler_params=pltpu.CompilerParams(needs_layout_passes=False,
                                         use_tc_tiling_on_sc=False),
    scratch_types=[
        pltpu.VMEM((EPT,), jnp.int32),
        pltpu.VMEM((NCHP, GC), jnp.int32),
        pltpu.VMEM((NBUF, GC, H), jnp.float32),
        pltpu.VMEM((NBUF, GC, H), jnp.float32),
        pltpu.VMEM((NBUF, GC, AW), jnp.float32),
        pltpu.VMEM((N,), jnp.float32),
        pltpu.VMEM((H,), jnp.float32),
        pltpu.VMEM_SHARED((N, AW), jnp.float32),
    ] + [pltpu.SemaphoreType.DMA] * (3 * NBUF),
)
def _gate_edge_sc(xa_hbm, ea_hbm, src_hbm, dst2_hbm, ad_hbm, attl_hbm,
                  out_hbm, src_v, dst_v, rows_v, ea_v, msg_v,
                  ad_v, attl_v, acc_sh, *sems):
    cid = lax.axis_index("c")
    sid = lax.axis_index("s")
    wid = sid * NC + cid
    base = wid * EPT

    _zero_msg(msg_v.at[0], GC, AW)
    _zero_acc_stripe(msg_v.at[0], acc_sh, sid)

    pltpu.sync_copy(src_hbm.at[pl.ds(base, EPT)], src_v)
    pltpu.sync_copy(dst2_hbm.at[wid], dst_v)
    pltpu.sync_copy(ad_hbm, ad_v)
    pltpu.sync_copy(attl_hbm, attl_v)
    plsc.subcore_barrier()

    iota = lax.iota(jnp.int32, L)
    sems_e = sems[:NBUF]
    sems_g = sems[NBUF:2 * NBUF]
    sems_s = sems[2 * NBUF:]

    def issue(ch, sl):
        pltpu.async_copy(ea_hbm.at[pl.ds(base + ch * GC, GC)], ea_v.at[sl],
                         sems_e[sl])
        pltpu.async_copy(xa_hbm.at[src_v.at[pl.ds(ch * GC, GC)]],
                         rows_v.at[sl], sems_g[sl])

    def wait_slot(sl):
        pltpu.make_async_copy(ea_hbm.at[pl.ds(base, GC)], ea_v.at[sl],
                              sems_e[sl]).wait()
        pltpu.make_async_copy(ea_hbm.at[pl.ds(base, GC)], rows_v.at[sl],
                              sems_g[sl]).wait()

    def work(ch, sl):
        def grp(g, _):
            o = pl.multiple_of(g * L, L)
            d16 = dst_v[ch, pl.ds(o, L)]
            adv = plsc.load_gather(ad_v, [d16])
            for lane in range(L):
                e = o + lane
                tv = jnp.zeros((L,), jnp.float32)
                hjs = []
                for j in range(H // L):
                    sj = pl.ds(j * L, L)
                    v = rows_v[sl, e, sj] + ea_v[sl, e, sj]
                    hj = jnp.maximum(v, 0.01 * v)
                    hjs.append(hj)
                    tv = tv + hj * attl_v[sj]
                t = jnp.sum(tv) + adv[lane]
                av = jnp.full((L,), t, jnp.float32)
                exv = jnp.exp(jnp.maximum(av, 0.01 * av))
                for j in range(H // L):
                    msg_v[sl, e, pl.ds(j * L, L)] = hjs[j] * exv
                msg_v[sl, e, pl.ds(H, L)] = jnp.where(iota == 0, exv, 0.0)
            return 0

        lax.fori_loop(0, GC // L, grp, 0)
        pltpu.async_copy(msg_v.at[sl], acc_sh.at[dst_v.at[ch]], sems_s[sl],
                         add=True)

    def wait_scat(sl):
        pltpu.make_async_copy(msg_v.at[sl], acc_sh.at[dst_v.at[0]],
                              sems_s[sl]).wait()

    for c0 in range(NBUF - 1):
        issue(c0, c0)

    def chunk_body(ch, _):
        for par in range(NBUF):
            @pl.when(lax.rem(ch, NBUF) == par)
            def _():
                @pl.when(ch + NBUF - 1 < NCH)
                def _():
                    issue(ch + NBUF - 1, (par + NBUF - 1) % NBUF)
                wait_slot(par)

                @pl.when(ch >= NBUF)
                def _():
                    wait_scat(par)
                work(ch, par)
        return 0

    lax.fori_loop(0, NCH, chunk_body, 0)
    for sl in range(NBUF):
        wait_scat(sl)
    plsc.subcore_barrier()
    pltpu.sync_copy(acc_sh.at[pl.ds(sid * RPT, RPT)], out_hbm.at[cid, sid])


@functools.partial(
    pl.kernel,
    out_type=jax.ShapeDtypeStruct((NC, NS, RPT, AW), jnp.float32),
    mesh=_mesh,
    compiler_params=pltpu.CompilerParams(needs_layout_passes=False,
                                         use_tc_tiling_on_sc=False),
    scratch_types=[
        pltpu.VMEM((EPT,), jnp.int32),
        pltpu.VMEM((NCHP, GC), jnp.int32),
        pltpu.VMEM((NBUF, GC, H), jnp.float32),
        pltpu.VMEM((NBUF, GC, AW), jnp.float32),
        pltpu.VMEM((N,), jnp.float32),
        pltpu.VMEM((N,), jnp.float32),
        pltpu.VMEM_SHARED((N, AW), jnp.float32),
    ] + [pltpu.SemaphoreType.DMA] * (2 * NBUF),
)
def _att_edge_sc(x_hbm, src_hbm, dst2_hbm, as_hbm, ad_hbm, out_hbm,
                 src_v, dst_v, rows_v, msg_v, as_v, ad_v, acc_sh, *sems):
    cid = lax.axis_index("c")
    sid = lax.axis_index("s")
    wid = sid * NC + cid
    base = wid * EPT

    _zero_msg(msg_v.at[0], GC, AW)
    _zero_acc_stripe(msg_v.at[0], acc_sh, sid)

    pltpu.sync_copy(src_hbm.at[pl.ds(base, EPT)], src_v)
    pltpu.sync_copy(dst2_hbm.at[wid], dst_v)
    pltpu.sync_copy(as_hbm, as_v)
    pltpu.sync_copy(ad_hbm, ad_v)
    plsc.subcore_barrier()

    iota = lax.iota(jnp.int32, L)
    sems_g = sems[:NBUF]
    sems_s = sems[NBUF:]

    def issue(ch, sl):
        pltpu.async_copy(x_hbm.at[src_v.at[pl.ds(ch * GC, GC)]],
                         rows_v.at[sl], sems_g[sl])

    def wait_slot(sl):
        pltpu.make_async_copy(x_hbm.at[pl.ds(0, GC)], rows_v.at[sl],
                              sems_g[sl]).wait()

    def work(ch, sl):
        def grp(g, _):
            o = pl.multiple_of(g * L, L)
            s16 = src_v[pl.ds(pl.multiple_of(ch * GC + g * L, L), L)]
            d16 = dst_v[ch, pl.ds(o, L)]
            a = plsc.load_gather(as_v, [s16]) + plsc.load_gather(ad_v, [d16])
            exv16 = jnp.exp(jnp.maximum(a, 0.01 * a))
            for lane in range(L):
                e = o + lane
                exv = jnp.full((L,), exv16[lane], jnp.float32)
                for j in range(H // L):
                    sj = pl.ds(j * L, L)
                    msg_v[sl, e, sj] = rows_v[sl, e, sj] * exv
                msg_v[sl, e, pl.ds(H, L)] = jnp.where(iota == 0, exv, 0.0)
            return 0

        lax.fori_loop(0, GC // L, grp, 0)
        pltpu.async_copy(msg_v.at[sl], acc_sh.at[dst_v.at[ch]], sems_s[sl],
                         add=True)

    def wait_scat(sl):
        pltpu.make_async_copy(msg_v.at[sl], acc_sh.at[dst_v.at[0]],
                              sems_s[sl]).wait()

    for c0 in range(NBUF - 1):
        issue(c0, c0)

    def chunk_body(ch, _):
        for par in range(NBUF):
            @pl.when(lax.rem(ch, NBUF) == par)
            def _():
                @pl.when(ch + NBUF - 1 < NCH)
                def _():
                    issue(ch + NBUF - 1, (par + NBUF - 1) % NBUF)
                wait_slot(par)

                @pl.when(ch >= NBUF)
                def _():
                    wait_scat(par)
                work(ch, par)
        return 0

    lax.fori_loop(0, NCH, chunk_body, 0)
    for sl in range(NBUF):
        wait_scat(sl)
    plsc.subcore_barrier()
    pltpu.sync_copy(acc_sh.at[pl.ds(sid * RPT, RPT)], out_hbm.at[cid, sid])


@functools.partial(
    pl.kernel,
    out_type=jax.ShapeDtypeStruct((NW, B + 1, AW), jnp.float32),
    mesh=_mesh,
    compiler_params=pltpu.CompilerParams(needs_layout_passes=False, use_tc_tiling_on_sc=False),
    scratch_types=[
        pltpu.VMEM((RP3, H), jnp.float32),
        pltpu.VMEM((RP3,), jnp.int32),
        pltpu.VMEM((RP3,), jnp.float32),
        pltpu.VMEM((B + 16,), jnp.float32),
        pltpu.VMEM((B + 1, AW), jnp.float32),
    ],
)
def _pool_att_sc(x_hbm, b_hbm, as_hbm, adt_hbm, out_hbm,
                 rows_v, b_v, as_v, adt_v, acc_v):
    cid = lax.axis_index("c")
    sid = lax.axis_index("s")
    wid = sid * NC + cid
    base = wid * RP3

    z = jnp.zeros((L,), jnp.float32)

    def zrow(i, _):
        for j in range(AW // L):
            acc_v[i, pl.ds(j * L, L)] = z
        return 0

    lax.fori_loop(0, B + 1, zrow, 0)

    pltpu.sync_copy(x_hbm.at[pl.ds(base, RP3)], rows_v)
    pltpu.sync_copy(b_hbm.at[pl.ds(base, RP3)], b_v)
    pltpu.sync_copy(as_hbm.at[pl.ds(base, RP3)], as_v)
    pltpu.sync_copy(adt_hbm, adt_v)

    iota = lax.iota(jnp.int32, L)

    def pg(g, _):
        o = pl.multiple_of(g * L, L)
        b16 = b_v[pl.ds(o, L)]
        a = as_v[pl.ds(o, L)] + plsc.load_gather(adt_v, [b16])
        exv16 = jnp.exp(jnp.maximum(a, 0.01 * a))
        for lane in range(L):
            e = o + lane
            de = b16[lane]
            exv = jnp.full((L,), exv16[lane], jnp.float32)
            for j in range(H // L):
                sl = pl.ds(j * L, L)
                acc_v[de, sl] = acc_v[de, sl] + rows_v[e, sl] * exv
            sl = pl.ds(H, L)
            acc_v[de, sl] = acc_v[de, sl] + jnp.where(iota == 0, exv, 0.0)
        return 0

    lax.fori_loop(0, RP3 // L, pg, 0)

    pltpu.sync_copy(acc_v, out_hbm.at[wid])


NB = 1000
NGRID = N // NB


def _full(spec_shape):
    nd = len(spec_shape)
    return pl.BlockSpec(spec_shape, lambda *_: (0,) * nd)


def _gru_block(xv, h, Wih, Whh, bih, bhh):
    gi = jnp.dot(xv, Wih.T, preferred_element_type=jnp.float32) + bih
    gh = jnp.dot(h, Whh.T, preferred_element_type=jnp.float32) + bhh
    r = jax.nn.sigmoid(gi[:, :H] + gh[:, :H])
    z = jax.nn.sigmoid(gi[:, H:2 * H] + gh[:, H:2 * H])
    n_ = jnp.tanh(gi[:, 2 * H:] + r * gh[:, 2 * H:])
    return (1.0 - z) * n_ + z * h


def _elu(v):
    return jnp.where(v > 0, v, jnp.exp(jnp.minimum(v, 0.0)) - 1.0)


def _tk_node_prep(x, W1, b1, W1a, attr):
    def body(x_r, W1_r, b1_r, W1a_r, attr_r, x1_r, xa_r, adg_r):
        x1 = _lk(jnp.dot(x_r[...], W1_r[...].T,
                         preferred_element_type=jnp.float32) + b1_r[...])
        x1_r[...] = x1
        xa_r[...] = jnp.dot(x1, W1a_r[...].T,
                            preferred_element_type=jnp.float32)
        adg_r[...] = jnp.sum(x1 * attr_r[...].T, axis=1, keepdims=True)

    return pl.pallas_call(
        body,
        grid=(NGRID,),
        in_specs=[pl.BlockSpec((NB, 128), lambda i: (i, 0)),
                  _full((H, 128)), _full((H,)), _full((H, H)),
                  _full((H, 1))],
        out_specs=[pl.BlockSpec((NB, H), lambda i: (i, 0)),
                   pl.BlockSpec((NB, H), lambda i: (i, 0)),
                   pl.BlockSpec((NB, 1), lambda i: (i, 0))],
        out_shape=[jax.ShapeDtypeStruct((N, H), jnp.float32),
                   jax.ShapeDtypeStruct((N, H), jnp.float32),
                   jax.ShapeDtypeStruct((N, 1), jnp.float32)],
    )(x, W1, b1, W1a, attr)


EB = 4000


def _tk_ea(edge_attr, W1b):
    def body(e_r, w_r, o_r):
        o_r[...] = jnp.dot(e_r[...], w_r[...].T,
                           preferred_element_type=jnp.float32)

    return pl.pallas_call(
        body,
        grid=(E // EB,),
        in_specs=[pl.BlockSpec((EB, 16), lambda i: (i, 0)),
                  _full((H, 16))],
        out_specs=pl.BlockSpec((EB, H), lambda i: (i, 0)),
        out_shape=jax.ShapeDtypeStruct((E, H), jnp.float32),
    )(edge_attr, W1b)


def _tk_conv_post(acc, xprev, Wc, bc, Wih, Whh, bih, bhh, Wn, asv, adv):
    def body(a0_r, a1_r, xp_r, Wc_r, bc_r, Wih_r, Whh_r, bih_r, bhh_r,
             Wn_r, as0_r, ad0_r, xn_r, as_r, ad_r):
        a = a0_r[0] + a1_r[0]
        u = a[:, :H] / (a[:, H:H + 1] + 1e-16)
        h = _elu(jnp.dot(u, Wc_r[...].T,
                         preferred_element_type=jnp.float32) + bc_r[...])
        xn = jnp.maximum(
            _gru_block(h, xp_r[...], Wih_r[...], Whh_r[...], bih_r[...],
                       bhh_r[...]), 0.0)
        xn_r[...] = xn
        ws = jnp.dot(Wn_r[...].T, as0_r[...],
                     preferred_element_type=jnp.float32)
        wd = jnp.dot(Wn_r[...].T, ad0_r[...],
                     preferred_element_type=jnp.float32)
        as_r[...] = jnp.sum(xn * ws.T, axis=1, keepdims=True)
        ad_r[...] = jnp.sum(xn * wd.T, axis=1, keepdims=True)

    return pl.pallas_call(
        body,
        grid=(NGRID,),
        in_specs=[pl.BlockSpec((1, NB, AW), lambda i: (0, i, 0)),
                  pl.BlockSpec((1, NB, AW), lambda i: (1, i, 0)),
                  pl.BlockSpec((NB, H), lambda i: (i, 0)),
                  _full((H, H)), _full((H,)),
                  _full((3 * H, H)), _full((3 * H, H)),
                  _full((3 * H,)), _full((3 * H,)),
                  _full((H, H)), _full((H, 1)), _full((H, 1))],
        out_specs=[pl.BlockSpec((NB, H), lambda i: (i, 0)),
                   pl.BlockSpec((NB, 1), lambda i: (i, 0)),
                   pl.BlockSpec((NB, 1), lambda i: (i, 0))],
        out_shape=[jax.ShapeDtypeStruct((N, H), jnp.float32),
                   jax.ShapeDtypeStruct((N, 1), jnp.float32),
                   jax.ShapeDtypeStruct((N, 1), jnp.float32)],
    )(acc, acc, xprev, Wc, bc, Wih, Whh, bih, bhh, Wn, asv, adv)


def _tk_pool_post(p32, Wmc, attd):
    def body(p_r, W_r, ad_r, og_r, adg_r):
        seg = jnp.sum(p_r[...], axis=0)[:B, :H]
        og = jnp.maximum(seg, 0.0)
        og_r[...] = og
        wv = jnp.dot(W_r[...].T, ad_r[...],
                     preferred_element_type=jnp.float32)
        adg_r[...] = jnp.sum(og * wv.T, axis=1, keepdims=True)

    return pl.pallas_call(
        body,
        in_specs=[_full((NW, B + 1, AW)), _full((H, H)), _full((H, 1))],
        out_specs=[_full((B, H)), _full((B, 1))],
        out_shape=[jax.ShapeDtypeStruct((B, H), jnp.float32),
                   jax.ShapeDtypeStruct((B, 1), jnp.float32)],
    )(p32, Wmc, attd)


def _tk_mc_post(m32, og, Wmc, bmc, Wih, Whh, bih, bhh, attd):
    def body(m_r, og_r, W_r, b_r, Wih_r, Whh_r, bih_r, bhh_r, ad_r,
             on_r, adg_r):
        a = jnp.sum(m_r[...], axis=0)[:B]
        u = a[:, :H] / (a[:, H:H + 1] + 1e-16)
        h = _elu(jnp.dot(u, W_r[...].T,
                         preferred_element_type=jnp.float32) + b_r[...])
        on = jnp.maximum(
            _gru_block(h, og_r[...], Wih_r[...], Whh_r[...], bih_r[...],
                       bhh_r[...]), 0.0)
        on_r[...] = on
        wv = jnp.dot(W_r[...].T, ad_r[...],
                     preferred_element_type=jnp.float32)
        adg_r[...] = jnp.sum(on * wv.T, axis=1, keepdims=True)

    return pl.pallas_call(
        body,
        in_specs=[_full((NW, B + 1, AW)), _full((B, H)),
                  _full((H, H)), _full((H,)),
                  _full((3 * H, H)), _full((3 * H, H)),
                  _full((3 * H,)), _full((3 * H,)), _full((H, 1))],
        out_specs=[_full((B, H)), _full((B, 1))],
        out_shape=[jax.ShapeDtypeStruct((B, H), jnp.float32),
                   jax.ShapeDtypeStruct((B, 1), jnp.float32)],
    )(m32, og, Wmc, bmc, Wih, Whh, bih, bhh, attd)


GB = 64
GL = 3072


def _tk_gene(gene, gc_W, gc_b):
    def body(g_r, w_r, b_r, o_r):
        m_idx = lax.broadcasted_iota(jnp.int32, (GL, H), 0)
        i_idx = lax.broadcasted_iota(jnp.int32, (GL, H), 1)
        mask = (m_idx // (GL // H) == i_idx).astype(jnp.float32) / 16.0
        rem = m_idx % 3
        acc = jnp.zeros((GB, H), jnp.float32)
        w = w_r[...]
        for c in range(4):
            vals = jnp.where(rem == 0, w[c, 0],
                             jnp.where(rem == 1, w[c, 1], w[c, 2]))
            acc = acc + jnp.dot(g_r[0, :, c, :], vals * mask,
                                preferred_element_type=jnp.float32)
        o_r[...] = acc + b_r[0]

    return pl.pallas_call(
        body,
        grid=(B // GB,),
        in_specs=[pl.BlockSpec((1, GB, 4, GL), lambda i: (0, i, 0, 0)),
                  _full((4, 3)), _full((1,))],
        out_specs=pl.BlockSpec((GB, H), lambda i: (i, 0)),
        out_shape=jax.ShapeDtypeStruct((B, H), jnp.float32),
    )(gene[None], gc_W, gc_b)


def _tk_final(og, gp, taxonomy, duration, W_dur, b_dur, W4, b4, W5, b5):
    def body(og_r, gp_r, tx_r, du_r, Wd_r, bd_r, W4_r, b4_r, W5_r, b5_r,
             o_r):
        dur = jnp.maximum(
            jnp.dot(du_r[...], Wd_r[...].T,
                    preferred_element_type=jnp.float32) + bd_r[...], 0.0)
        cat = jnp.concatenate([og_r[...], gp_r[...], tx_r[...], dur], 1)
        c4 = jnp.dot(cat, W4_r[...].T,
                     preferred_element_type=jnp.float32) + b4_r[...]
        o_r[...] = jnp.sum(c4 * W5_r[...], axis=1, keepdims=True) + b5_r[0]

    return pl.pallas_call(
        body,
        in_specs=[_full((B, H)), _full((B, H)), _full((B, H)),
                  _full((B, DUR)), _full((H, DUR)), _full((H,)),
                  _full((H, 4 * H)), _full((H,)),
                  _full((1, H)), _full((1,))],
        out_specs=_full((B, 1)),
        out_shape=jax.ShapeDtypeStruct((B, 1), jnp.float32),
    )(og, gp, taxonomy, duration, W_dur, b_dur, W4, b4, W5, b5)


def _gru(xv, h, Wih, Whh, bih, bhh):
    gi = xv @ Wih.T + bih
    gh = h @ Whh.T + bhh
    ir, iz, inn = jnp.split(gi, 3, axis=1)
    hr, hz, hn = jnp.split(gh, 3, axis=1)
    r = jax.nn.sigmoid(ir + hr)
    zz = jax.nn.sigmoid(iz + hz)
    n_ = jnp.tanh(inn + r * hn)
    return (1.0 - zz) * n_ + zz * h


def kernel(x, edge_index, edge_attr, batch, gene, taxonomy, duration,
           W1, b1, ge_W1, ge_W2, ge_att_l, ge_att_r, ge_bias,
           gru_Wih, gru_Whh, gru_bih, gru_bhh,
           ac_W, ac_att_src, ac_att_dst, ac_bias,
           ag_Wih, ag_Whh, ag_bih, ag_bhh,
           mc_W, mc_att_src, mc_att_dst, mc_bias,
           mg_Wih, mg_Whh, mg_bih, mg_bhh,
           gc_W, gc_b, W_dur, b_dur, W4, b4, W5, b5):
    f32 = jnp.float32
    src = edge_index[0]
    dst = edge_index[1]
    dst2 = jnp.pad(dst.reshape(NW, NCH, GC), ((0, 0), (0, NCHP - NCH), (0, 0)))

    W1a = ge_W1[:, :H]
    W1b = ge_W1[:, H:]
    x1, xa, adg = _tk_node_prep(x, W1, b1, W1a, ge_att_r.reshape(H, 1))
    ea = _tk_ea(edge_attr, W1b)

    acc = _gate_edge_sc(xa, ea, src, dst2, adg.reshape(N), ge_att_l)
    acc = acc.reshape(NC, N, AW)
    x2, as2, ad2 = _tk_conv_post(acc, x1, ge_W2, ge_bias,
                                 gru_Wih, gru_Whh, gru_bih, gru_bhh,
                                 ac_W, ac_att_src.reshape(H, 1),
                                 ac_att_dst.reshape(H, 1))

    acc2 = _att_edge_sc(x2, src, dst2, as2.reshape(N),
                        ad2.reshape(N)).reshape(NC, N, AW)
    x3, as3, _ = _tk_conv_post(acc2, x2, ac_W, ac_bias,
                               ag_Wih, ag_Whh, ag_bih, ag_bhh,
                               mc_W, mc_att_src.reshape(H, 1),
                               mc_att_src.reshape(H, 1))

    x3p = jnp.concatenate([x3, jnp.zeros((NP - N, H), f32)], 0)
    bp = jnp.concatenate([batch, jnp.full((NP - N,), B, jnp.int32)], 0)
    as3p = jnp.concatenate([as3.reshape(N), jnp.zeros((NP - N,), f32)], 0)
    zs = jnp.zeros((NP,), f32)
    zt = jnp.zeros((B + 16,), f32)

    p32 = _pool_att_sc(x3p, bp, zs, zt)
    out_g, adg1 = _tk_pool_post(p32, mc_W, mc_att_dst.reshape(H, 1))

    adgp = adg1
    for _ in range(2):
        m32 = _pool_att_sc(x3p, bp, as3p,
                           jnp.pad(adgp.reshape(B), (0, 16)))
        out_g, adgp = _tk_mc_post(m32, out_g, mc_W, mc_bias,
                                  mg_Wih, mg_Whh, mg_bih, mg_bhh,
                                  mc_att_dst.reshape(H, 1))

    gp = _tk_gene(gene, gc_W, gc_b)
    return _tk_final(out_g, gp, taxonomy, duration, W_dur, b_dur,
                     W4, b4, W5, b5)


# speedup vs baseline: 1.3001x; 1.1486x over previous
import functools

import jax
import jax.numpy as jnp
from jax import lax
from jax.experimental import pallas as pl
from jax.experimental.pallas import tpu as pltpu
from jax.experimental.pallas import tpu_sc as plsc

NC, NS, L = 2, 16, 16
NW = NC * NS

N = 10000
E = 320000
H = 64
B = 512
AW = 80
EPT = E // NW
GC = 80
NCH = EPT // GC
RPT = N // NS
NP = 10240
DUR = 8
RP3 = NP // NW

_mesh = plsc.VectorSubcoreMesh(core_axis_name="c", subcore_axis_name="s")


def _lk(v):
    return jnp.maximum(v, 0.01 * v)


def _zero_msg(msg_v, nrow, width):
    z = jnp.zeros((L,), jnp.float32)

    def zrow(i, _):
        for j in range(width // L):
            msg_v[i, pl.ds(j * L, L)] = z
        return 0

    lax.fori_loop(0, nrow, zrow, 0)


def _zero_acc_stripe(msg_v, acc_sh, sid):
    off = 0
    for nblk in (80, 80, 80, 80, 80, 80, 80, 65):
        pltpu.sync_copy(msg_v.at[pl.ds(0, nblk)],
                        acc_sh.at[pl.ds(sid * RPT + off, nblk)])
        off += nblk


NCHP = 128
NBUF = 2


@functools.partial(
    pl.kernel,
    out_type=jax.ShapeDtypeStruct((NC, NS, RPT, AW), jnp.float32),
    mesh=_mesh,
    compiler_params=pltpu.CompilerParams(needs_layout_passes=False,
                                         use_tc_tiling_on_sc=False),
    scratch_types=[
        pltpu.VMEM((EPT,), jnp.int32),
        pltpu.VMEM((NCHP, GC), jnp.int32),
        pltpu.VMEM((NBUF, GC, H), jnp.float32),
        pltpu.VMEM((NBUF, GC, 2 * H), jnp.float32),
        pltpu.VMEM((NBUF, GC, AW), jnp.float32),
        pltpu.VMEM((N,), jnp.float32),
        pltpu.VMEM((H,), jnp.float32),
        pltpu.VMEM_SHARED((N, AW), jnp.float32),
    ] + [pltpu.SemaphoreType.DMA] * (3 * NBUF),
)
def _gate_edge_sc(xa_hbm, ea_hbm, src_hbm, dst2_hbm, ad_hbm, attl_hbm,
                  out_hbm, src_v, dst_v, rows_v, ea_v, msg_v,
                  ad_v, attl_v, acc_sh, *sems):
    cid = lax.axis_index("c")
    sid = lax.axis_index("s")
    wid = sid * NC + cid
    base = wid * EPT
    erow = lax.rem(base, E // 2)
    ecol = (base // (E // 2)) * H

    _zero_msg(msg_v.at[0], GC, AW)
    _zero_acc_stripe(msg_v.at[0], acc_sh, sid)

    pltpu.sync_copy(src_hbm.at[pl.ds(base, EPT)], src_v)
    pltpu.sync_copy(dst2_hbm.at[wid], dst_v)
    pltpu.sync_copy(ad_hbm, ad_v)
    pltpu.sync_copy(attl_hbm, attl_v)
    plsc.subcore_barrier()

    iota = lax.iota(jnp.int32, L)
    sems_e = sems[:NBUF]
    sems_g = sems[NBUF:2 * NBUF]
    sems_s = sems[2 * NBUF:]

    def issue(ch, sl):
        pltpu.async_copy(ea_hbm.at[pl.ds(erow + ch * GC, GC)],
                         ea_v.at[sl], sems_e[sl])
        pltpu.async_copy(xa_hbm.at[src_v.at[pl.ds(ch * GC, GC)]],
                         rows_v.at[sl], sems_g[sl])

    def wait_slot(sl):
        pltpu.make_async_copy(ea_hbm.at[pl.ds(0, GC)], ea_v.at[sl],
                              sems_e[sl]).wait()
        pltpu.make_async_copy(xa_hbm.at[pl.ds(0, GC)], rows_v.at[sl],
                              sems_g[sl]).wait()

    def work(ch, sl):
        def grp(g, _):
            o = pl.multiple_of(g * L, L)
            d16 = dst_v[ch, pl.ds(o, L)]
            adv = plsc.load_gather(ad_v, [d16])
            for lane in range(L):
                e = o + lane
                tv = jnp.zeros((L,), jnp.float32)
                hjs = []
                for j in range(H // L):
                    sj = pl.ds(j * L, L)
                    v = (rows_v[sl, e, sj]
                         + ea_v[sl, e, pl.ds(pl.multiple_of(
                               ecol + j * L, L), L)])
                    hj = jnp.maximum(v, 0.01 * v)
                    hjs.append(hj)
                    tv = tv + hj * attl_v[sj]
                t = jnp.sum(tv) + adv[lane]
                av = jnp.full((L,), t, jnp.float32)
                exv = jnp.exp(jnp.maximum(av, 0.01 * av))
                for j in range(H // L):
                    msg_v[sl, e, pl.ds(j * L, L)] = hjs[j] * exv
                msg_v[sl, e, pl.ds(H, L)] = jnp.where(iota == 0, exv, 0.0)
            return 0

        lax.fori_loop(0, GC // L, grp, 0)
        pltpu.async_copy(msg_v.at[sl], acc_sh.at[dst_v.at[ch]], sems_s[sl],
                         add=True)

    def wait_scat(sl):
        pltpu.make_async_copy(msg_v.at[sl], acc_sh.at[dst_v.at[0]],
                              sems_s[sl]).wait()

    for c0 in range(NBUF - 1):
        issue(c0, c0)

    def chunk_body(ch, _):
        for par in range(NBUF):
            @pl.when(lax.rem(ch, NBUF) == par)
            def _():
                @pl.when(ch + NBUF - 1 < NCH)
                def _():
                    issue(ch + NBUF - 1, (par + NBUF - 1) % NBUF)
                wait_slot(par)

                @pl.when(ch >= NBUF)
                def _():
                    wait_scat(par)
                work(ch, par)
        return 0

    lax.fori_loop(0, NCH, chunk_body, 0)
    for sl in range(NBUF):
        wait_scat(sl)
    plsc.subcore_barrier()
    pltpu.sync_copy(acc_sh.at[pl.ds(sid * RPT, RPT)], out_hbm.at[cid, sid])


@functools.partial(
    pl.kernel,
    out_type=jax.ShapeDtypeStruct((NC, NS, RPT, AW), jnp.float32),
    mesh=_mesh,
    compiler_params=pltpu.CompilerParams(needs_layout_passes=False,
                                         use_tc_tiling_on_sc=False),
    scratch_types=[
        pltpu.VMEM((EPT,), jnp.int32),
        pltpu.VMEM((NCHP, GC), jnp.int32),
        pltpu.VMEM((NBUF, GC, H), jnp.float32),
        pltpu.VMEM((NBUF, GC, AW), jnp.float32),
        pltpu.VMEM((N,), jnp.float32),
        pltpu.VMEM((N,), jnp.float32),
        pltpu.VMEM_SHARED((N, AW), jnp.float32),
    ] + [pltpu.SemaphoreType.DMA] * (2 * NBUF),
)
def _att_edge_sc(x_hbm, src_hbm, dst2_hbm, as_hbm, ad_hbm, out_hbm,
                 src_v, dst_v, rows_v, msg_v, as_v, ad_v, acc_sh, *sems):
    cid = lax.axis_index("c")
    sid = lax.axis_index("s")
    wid = sid * NC + cid
    base = wid * EPT

    _zero_msg(msg_v.at[0], GC, AW)
    _zero_acc_stripe(msg_v.at[0], acc_sh, sid)

    pltpu.sync_copy(src_hbm.at[pl.ds(base, EPT)], src_v)
    pltpu.sync_copy(dst2_hbm.at[wid], dst_v)
    pltpu.sync_copy(as_hbm, as_v)
    pltpu.sync_copy(ad_hbm, ad_v)
    plsc.subcore_barrier()

    iota = lax.iota(jnp.int32, L)
    sems_g = sems[:NBUF]
    sems_s = sems[NBUF:]

    def issue(ch, sl):
        pltpu.async_copy(x_hbm.at[src_v.at[pl.ds(ch * GC, GC)]],
                         rows_v.at[sl], sems_g[sl])

    def wait_slot(sl):
        pltpu.make_async_copy(x_hbm.at[pl.ds(0, GC)], rows_v.at[sl],
                              sems_g[sl]).wait()

    def work(ch, sl):
        def grp(g, _):
            o = pl.multiple_of(g * L, L)
            s16 = src_v[pl.ds(pl.multiple_of(ch * GC + g * L, L), L)]
            d16 = dst_v[ch, pl.ds(o, L)]
            a = plsc.load_gather(as_v, [s16]) + plsc.load_gather(ad_v, [d16])
            exv16 = jnp.exp(jnp.maximum(a, 0.01 * a))
            for lane in range(L):
                e = o + lane
                exv = jnp.full((L,), exv16[lane], jnp.float32)
                for j in range(H // L):
                    sj = pl.ds(j * L, L)
                    msg_v[sl, e, sj] = rows_v[sl, e, sj] * exv
                msg_v[sl, e, pl.ds(H, L)] = jnp.where(iota == 0, exv, 0.0)
            return 0

        lax.fori_loop(0, GC // L, grp, 0)
        pltpu.async_copy(msg_v.at[sl], acc_sh.at[dst_v.at[ch]], sems_s[sl],
                         add=True)

    def wait_scat(sl):
        pltpu.make_async_copy(msg_v.at[sl], acc_sh.at[dst_v.at[0]],
                              sems_s[sl]).wait()

    for c0 in range(NBUF - 1):
        issue(c0, c0)

    def chunk_body(ch, _):
        for par in range(NBUF):
            @pl.when(lax.rem(ch, NBUF) == par)
            def _():
                @pl.when(ch + NBUF - 1 < NCH)
                def _():
                    issue(ch + NBUF - 1, (par + NBUF - 1) % NBUF)
                wait_slot(par)

                @pl.when(ch >= NBUF)
                def _():
                    wait_scat(par)
                work(ch, par)
        return 0

    lax.fori_loop(0, NCH, chunk_body, 0)
    for sl in range(NBUF):
        wait_scat(sl)
    plsc.subcore_barrier()
    pltpu.sync_copy(acc_sh.at[pl.ds(sid * RPT, RPT)], out_hbm.at[cid, sid])


@functools.partial(
    pl.kernel,
    out_type=jax.ShapeDtypeStruct((NW, B + 1, AW), jnp.float32),
    mesh=_mesh,
    compiler_params=pltpu.CompilerParams(needs_layout_passes=False, use_tc_tiling_on_sc=False),
    scratch_types=[
        pltpu.VMEM((RP3, H), jnp.float32),
        pltpu.VMEM((RP3,), jnp.int32),
        pltpu.VMEM((RP3,), jnp.float32),
        pltpu.VMEM((B + 16,), jnp.float32),
        pltpu.VMEM((B + 1, AW), jnp.float32),
    ],
)
def _pool_att_sc(x_hbm, b_hbm, as_hbm, adt_hbm, out_hbm,
                 rows_v, b_v, as_v, adt_v, acc_v):
    cid = lax.axis_index("c")
    sid = lax.axis_index("s")
    wid = sid * NC + cid
    base = wid * RP3

    z = jnp.zeros((L,), jnp.float32)

    def zrow(i, _):
        for j in range(AW // L):
            acc_v[i, pl.ds(j * L, L)] = z
        return 0

    lax.fori_loop(0, B + 1, zrow, 0)

    pltpu.sync_copy(x_hbm.at[pl.ds(base, RP3)], rows_v)
    pltpu.sync_copy(b_hbm.at[pl.ds(base, RP3)], b_v)
    pltpu.sync_copy(as_hbm.at[pl.ds(base, RP3)], as_v)
    pltpu.sync_copy(adt_hbm, adt_v)

    iota = lax.iota(jnp.int32, L)

    def pg(g, _):
        o = pl.multiple_of(g * L, L)
        b16 = b_v[pl.ds(o, L)]
        a = as_v[pl.ds(o, L)] + plsc.load_gather(adt_v, [b16])
        exv16 = jnp.exp(jnp.maximum(a, 0.01 * a))
        for lane in range(L):
            e = o + lane
            de = b16[lane]
            exv = jnp.full((L,), exv16[lane], jnp.float32)
            for j in range(H // L):
                sl = pl.ds(j * L, L)
                acc_v[de, sl] = acc_v[de, sl] + rows_v[e, sl] * exv
            sl = pl.ds(H, L)
            acc_v[de, sl] = acc_v[de, sl] + jnp.where(iota == 0, exv, 0.0)
        return 0

    lax.fori_loop(0, RP3 // L, pg, 0)

    pltpu.sync_copy(acc_v, out_hbm.at[wid])


NB = 1000
NGRID = N // NB


def _full(spec_shape):
    nd = len(spec_shape)
    return pl.BlockSpec(spec_shape, lambda *_: (0,) * nd)


def _gru_block(xv, h, Wih, Whh, bih, bhh):
    gi = jnp.dot(xv, Wih.T, preferred_element_type=jnp.float32) + bih
    gh = jnp.dot(h, Whh.T, preferred_element_type=jnp.float32) + bhh
    r = jax.nn.sigmoid(gi[:, :H] + gh[:, :H])
    z = jax.nn.sigmoid(gi[:, H:2 * H] + gh[:, H:2 * H])
    n_ = jnp.tanh(gi[:, 2 * H:] + r * gh[:, 2 * H:])
    return (1.0 - z) * n_ + z * h


def _elu(v):
    return jnp.where(v > 0, v, jnp.exp(jnp.minimum(v, 0.0)) - 1.0)


def _tk_node_prep(x, W1, b1, W1a, attr):
    def body(x_r, W1_r, b1_r, W1a_r, attr_r, x1_r, xa_r, adg_r):
        x1 = _lk(jnp.dot(x_r[...], W1_r[...].T,
                         preferred_element_type=jnp.float32) + b1_r[...])
        x1_r[...] = x1
        xa_r[...] = jnp.dot(x1, W1a_r[...].T,
                            preferred_element_type=jnp.float32)
        adg_r[...] = jnp.sum(x1 * attr_r[...].T, axis=1, keepdims=True)

    return pl.pallas_call(
        body,
        grid=(NGRID,),
        in_specs=[pl.BlockSpec((NB, 128), lambda i: (i, 0)),
                  _full((H, 128)), _full((H,)), _full((H, H)),
                  _full((H, 1))],
        out_specs=[pl.BlockSpec((NB, H), lambda i: (i, 0)),
                   pl.BlockSpec((NB, H), lambda i: (i, 0)),
                   pl.BlockSpec((NB, 1), lambda i: (i, 0))],
        out_shape=[jax.ShapeDtypeStruct((N, H), jnp.float32),
                   jax.ShapeDtypeStruct((N, H), jnp.float32),
                   jax.ShapeDtypeStruct((N, 1), jnp.float32)],
    )(x, W1, b1, W1a, attr)


EB = 4000


def _tk_ea(edge_attr, W1b):
    def body(lo_r, hi_r, w_r, o_r):
        lo = jnp.dot(lo_r[...], w_r[...].T,
                     preferred_element_type=jnp.float32)
        hi = jnp.dot(hi_r[...], w_r[...].T,
                     preferred_element_type=jnp.float32)
        o_r[...] = jnp.concatenate([lo, hi], axis=1)

    return pl.pallas_call(
        body,
        grid=(E // 2 // EB,),
        in_specs=[pl.BlockSpec((EB, 16), lambda i: (i, 0)),
                  pl.BlockSpec((EB, 16), lambda i: (i + E // 2 // EB, 0)),
                  _full((H, 16))],
        out_specs=pl.BlockSpec((EB, 2 * H), lambda i: (i, 0)),
        out_shape=jax.ShapeDtypeStruct((E // 2, 2 * H), jnp.float32),
    )(edge_attr, edge_attr, W1b)


def _tk_conv_post(acc, xprev, Wc, bc, Wih, Whh, bih, bhh, Wn, asv, adv):
    def body(a0_r, a1_r, xp_r, Wc_r, bc_r, Wih_r, Whh_r, bih_r, bhh_r,
             Wn_r, as0_r, ad0_r, xn_r, as_r, ad_r):
        a = a0_r[0] + a1_r[0]
        u = a[:, :H] / (a[:, H:H + 1] + 1e-16)
        h = _elu(jnp.dot(u, Wc_r[...].T,
                         preferred_element_type=jnp.float32) + bc_r[...])
        xn = jnp.maximum(
            _gru_block(h, xp_r[...], Wih_r[...], Whh_r[...], bih_r[...],
                       bhh_r[...]), 0.0)
        xn_r[...] = xn
        ws = jnp.dot(Wn_r[...].T, as0_r[...],
                     preferred_element_type=jnp.float32)
        wd = jnp.dot(Wn_r[...].T, ad0_r[...],
                     preferred_element_type=jnp.float32)
        as_r[...] = jnp.sum(xn * ws.T, axis=1, keepdims=True)
        ad_r[...] = jnp.sum(xn * wd.T, axis=1, keepdims=True)

    return pl.pallas_call(
        body,
        grid=(NGRID,),
        in_specs=[pl.BlockSpec((1, NB, AW), lambda i: (0, i, 0)),
                  pl.BlockSpec((1, NB, AW), lambda i: (1, i, 0)),
                  pl.BlockSpec((NB, H), lambda i: (i, 0)),
                  _full((H, H)), _full((H,)),
                  _full((3 * H, H)), _full((3 * H, H)),
                  _full((3 * H,)), _full((3 * H,)),
                  _full((H, H)), _full((H, 1)), _full((H, 1))],
        out_specs=[pl.BlockSpec((NB, H), lambda i: (i, 0)),
                   pl.BlockSpec((NB, 1), lambda i: (i, 0)),
                   pl.BlockSpec((NB, 1), lambda i: (i, 0))],
        out_shape=[jax.ShapeDtypeStruct((N, H), jnp.float32),
                   jax.ShapeDtypeStruct((N, 1), jnp.float32),
                   jax.ShapeDtypeStruct((N, 1), jnp.float32)],
    )(acc, acc, xprev, Wc, bc, Wih, Whh, bih, bhh, Wn, asv, adv)


def _tk_pool_post(p32, Wmc, attd):
    def body(p_r, W_r, ad_r, og_r, adg_r):
        seg = jnp.sum(p_r[...], axis=0)[:B, :H]
        og = jnp.maximum(seg, 0.0)
        og_r[...] = og
        wv = jnp.dot(W_r[...].T, ad_r[...],
                     preferred_element_type=jnp.float32)
        adg_r[...] = jnp.sum(og * wv.T, axis=1, keepdims=True)

    return pl.pallas_call(
        body,
        in_specs=[_full((NW, B + 1, AW)), _full((H, H)), _full((H, 1))],
        out_specs=[_full((B, H)), _full((B, 1))],
        out_shape=[jax.ShapeDtypeStruct((B, H), jnp.float32),
                   jax.ShapeDtypeStruct((B, 1), jnp.float32)],
    )(p32, Wmc, attd)


def _tk_mc_post(m32, og, Wmc, bmc, Wih, Whh, bih, bhh, attd):
    def body(m_r, og_r, W_r, b_r, Wih_r, Whh_r, bih_r, bhh_r, ad_r,
             on_r, adg_r):
        a = jnp.sum(m_r[...], axis=0)[:B]
        u = a[:, :H] / (a[:, H:H + 1] + 1e-16)
        h = _elu(jnp.dot(u, W_r[...].T,
                         preferred_element_type=jnp.float32) + b_r[...])
        on = jnp.maximum(
            _gru_block(h, og_r[...], Wih_r[...], Whh_r[...], bih_r[...],
                       bhh_r[...]), 0.0)
        on_r[...] = on
        wv = jnp.dot(W_r[...].T, ad_r[...],
                     preferred_element_type=jnp.float32)
        adg_r[...] = jnp.sum(on * wv.T, axis=1, keepdims=True)

    return pl.pallas_call(
        body,
        in_specs=[_full((NW, B + 1, AW)), _full((B, H)),
                  _full((H, H)), _full((H,)),
                  _full((3 * H, H)), _full((3 * H, H)),
                  _full((3 * H,)), _full((3 * H,)), _full((H, 1))],
        out_specs=[_full((B, H)), _full((B, 1))],
        out_shape=[jax.ShapeDtypeStruct((B, H), jnp.float32),
                   jax.ShapeDtypeStruct((B, 1), jnp.float32)],
    )(m32, og, Wmc, bmc, Wih, Whh, bih, bhh, attd)


GB = 64
GL = 3072


def _tk_gene(gene, gc_W, gc_b):
    def body(g_r, w_r, b_r, o_r):
        m_idx = lax.broadcasted_iota(jnp.int32, (GL, H), 0)
        i_idx = lax.broadcasted_iota(jnp.int32, (GL, H), 1)
        mask = (m_idx // (GL // H) == i_idx).astype(jnp.float32) / 16.0
        rem = m_idx % 3
        acc = jnp.zeros((GB, H), jnp.float32)
        w = w_r[...]
        for c in range(4):
            vals = jnp.where(rem == 0, w[c, 0],
                             jnp.where(rem == 1, w[c, 1], w[c, 2]))
            acc = acc + jnp.dot(g_r[0, :, c, :], vals * mask,
                                preferred_element_type=jnp.float32)
        o_r[...] = acc + b_r[0]

    return pl.pallas_call(
        body,
        grid=(B // GB,),
        in_specs=[pl.BlockSpec((1, GB, 4, GL), lambda i: (0, i, 0, 0)),
                  _full((4, 3)), _full((1,))],
        out_specs=pl.BlockSpec((GB, H), lambda i: (i, 0)),
        out_shape=jax.ShapeDtypeStruct((B, H), jnp.float32),
    )(gene[None], gc_W, gc_b)


def _tk_final(og, gp, taxonomy, duration, W_dur, b_dur, W4, b4, W5, b5):
    def body(og_r, gp_r, tx_r, du_r, Wd_r, bd_r, W4_r, b4_r, W5_r, b5_r,
             o_r):
        dur = jnp.maximum(
            jnp.dot(du_r[...], Wd_r[...].T,
                    preferred_element_type=jnp.float32) + bd_r[...], 0.0)
        cat = jnp.concatenate([og_r[...], gp_r[...], tx_r[...], dur], 1)
        c4 = jnp.dot(cat, W4_r[...].T,
                     preferred_element_type=jnp.float32) + b4_r[...]
        o_r[...] = jnp.sum(c4 * W5_r[...], axis=1, keepdims=True) + b5_r[0]

    return pl.pallas_call(
        body,
        in_specs=[_full((B, H)), _full((B, H)), _full((B, H)),
                  _full((B, DUR)), _full((H, DUR)), _full((H,)),
                  _full((H, 4 * H)), _full((H,)),
                  _full((1, H)), _full((1,))],
        out_specs=_full((B, 1)),
        out_shape=jax.ShapeDtypeStruct((B, 1), jnp.float32),
    )(og, gp, taxonomy, duration, W_dur, b_dur, W4, b4, W5, b5)


def _gru(xv, h, Wih, Whh, bih, bhh):
    gi = xv @ Wih.T + bih
    gh = h @ Whh.T + bhh
    ir, iz, inn = jnp.split(gi, 3, axis=1)
    hr, hz, hn = jnp.split(gh, 3, axis=1)
    r = jax.nn.sigmoid(ir + hr)
    zz = jax.nn.sigmoid(iz + hz)
    n_ = jnp.tanh(inn + r * hn)
    return (1.0 - zz) * n_ + zz * h


def kernel(x, edge_index, edge_attr, batch, gene, taxonomy, duration,
           W1, b1, ge_W1, ge_W2, ge_att_l, ge_att_r, ge_bias,
           gru_Wih, gru_Whh, gru_bih, gru_bhh,
           ac_W, ac_att_src, ac_att_dst, ac_bias,
           ag_Wih, ag_Whh, ag_bih, ag_bhh,
           mc_W, mc_att_src, mc_att_dst, mc_bias,
           mg_Wih, mg_Whh, mg_bih, mg_bhh,
           gc_W, gc_b, W_dur, b_dur, W4, b4, W5, b5):
    f32 = jnp.float32
    src = edge_index[0]
    dst = edge_index[1]
    dst2 = jnp.pad(dst.reshape(NW, NCH, GC), ((0, 0), (0, NCHP - NCH), (0, 0)))

    W1a = ge_W1[:, :H]
    W1b = ge_W1[:, H:]
    x1, xa, adg = _tk_node_prep(x, W1, b1, W1a, ge_att_r.reshape(H, 1))
    ea = _tk_ea(edge_attr, W1b)

    acc = _gate_edge_sc(xa, ea, src, dst2, adg.reshape(N), ge_att_l)
    acc = acc.reshape(NC, N, AW)
    x2, as2, ad2 = _tk_conv_post(acc, x1, ge_W2, ge_bias,
                                 gru_Wih, gru_Whh, gru_bih, gru_bhh,
                                 ac_W, ac_att_src.reshape(H, 1),
                                 ac_att_dst.reshape(H, 1))

    acc2 = _att_edge_sc(x2, src, dst2, as2.reshape(N),
                        ad2.reshape(N)).reshape(NC, N, AW)
    x3, as3, _ = _tk_conv_post(acc2, x2, ac_W, ac_bias,
                               ag_Wih, ag_Whh, ag_bih, ag_bhh,
                               mc_W, mc_att_src.reshape(H, 1),
                               mc_att_src.reshape(H, 1))

    x3p = jnp.concatenate([x3, jnp.zeros((NP - N, H), f32)], 0)
    bp = jnp.concatenate([batch, jnp.full((NP - N,), B, jnp.int32)], 0)
    as3p = jnp.concatenate([as3.reshape(N), jnp.zeros((NP - N,), f32)], 0)
    zs = jnp.zeros((NP,), f32)
    zt = jnp.zeros((B + 16,), f32)

    p32 = _pool_att_sc(x3p, bp, zs, zt)
    out_g, adg1 = _tk_pool_post(p32, mc_W, mc_att_dst.reshape(H, 1))

    adgp = adg1
    for _ in range(2):
        m32 = _pool_att_sc(x3p, bp, as3p,
                           jnp.pad(adgp.reshape(B), (0, 16)))
        out_g, adgp = _tk_mc_post(m32, out_g, mc_W, mc_bias,
                                  mg_Wih, mg_Whh, mg_bih, mg_bhh,
                                  mc_att_dst.reshape(H, 1))

    gp = _tk_gene(gene, gc_W, gc_b)
    return _tk_final(out_g, gp, taxonomy, duration, W_dur, b_dur,
                     W4, b4, W5, b5)


# speedup vs baseline: 1.4179x; 1.0906x over previous
import functools

import jax
import jax.numpy as jnp
from jax import lax
from jax.experimental import pallas as pl
from jax.experimental.pallas import tpu as pltpu
from jax.experimental.pallas import tpu_sc as plsc

NC, NS, L = 2, 16, 16
NW = NC * NS

N = 10000
E = 320000
H = 64
B = 512
AW = 80
EPT = E // NW
GC = 80
NCH = EPT // GC
RPT = N // NS
NP = 10240
DUR = 8
RP3 = NP // NW

_mesh = plsc.VectorSubcoreMesh(core_axis_name="c", subcore_axis_name="s")


def _lk(v):
    return jnp.maximum(v, 0.01 * v)


def _zero_msg(msg_v, nrow, width):
    z = jnp.zeros((L,), jnp.float32)

    def zrow(i, _):
        for j in range(width // L):
            msg_v[i, pl.ds(j * L, L)] = z
        return 0

    lax.fori_loop(0, nrow, zrow, 0)


def _zero_acc_stripe(msg_v, acc_sh, sid):
    off = 0
    for nblk in (80, 80, 80, 80, 80, 80, 80, 65):
        pltpu.sync_copy(msg_v.at[pl.ds(0, nblk)],
                        acc_sh.at[pl.ds(sid * RPT + off, nblk)])
        off += nblk


NCHP = 128
NBUF = 2


@functools.partial(
    pl.kernel,
    out_type=jax.ShapeDtypeStruct((NC, NS, RPT, AW), jnp.float32),
    mesh=_mesh,
    compiler_params=pltpu.CompilerParams(needs_layout_passes=False,
                                         use_tc_tiling_on_sc=False),
    scratch_types=[
        pltpu.VMEM((EPT,), jnp.int32),
        pltpu.VMEM((NCHP, GC), jnp.int32),
        pltpu.VMEM((NBUF, GC, H), jnp.float32),
        pltpu.VMEM((NBUF, GC, 2 * H), jnp.float32),
        pltpu.VMEM((NBUF, GC, AW), jnp.float32),
        pltpu.VMEM((N,), jnp.float32),
        pltpu.VMEM((H,), jnp.float32),
        pltpu.VMEM_SHARED((N, AW), jnp.float32),
    ] + [pltpu.SemaphoreType.DMA] * (3 * NBUF),
)
def _gate_edge_sc(xa_hbm, ea_hbm, src_hbm, dst2_hbm, ad_hbm, attl_hbm,
                  out_hbm, src_v, dst_v, rows_v, ea_v, msg_v,
                  ad_v, attl_v, acc_sh, *sems):
    cid = lax.axis_index("c")
    sid = lax.axis_index("s")
    wid = sid * NC + cid
    base = wid * EPT
    erow = lax.rem(base, E // 2)
    ecol = (base // (E // 2)) * H

    _zero_msg(msg_v.at[0], GC, AW)
    _zero_acc_stripe(msg_v.at[0], acc_sh, sid)

    pltpu.sync_copy(src_hbm.at[pl.ds(base, EPT)], src_v)
    pltpu.sync_copy(dst2_hbm.at[wid], dst_v)
    pltpu.sync_copy(ad_hbm, ad_v)
    pltpu.sync_copy(attl_hbm, attl_v)
    plsc.subcore_barrier()

    iota = lax.iota(jnp.int32, L)
    sems_e = sems[:NBUF]
    sems_g = sems[NBUF:2 * NBUF]
    sems_s = sems[2 * NBUF:]

    def issue(ch, sl):
        pltpu.async_copy(ea_hbm.at[pl.ds(erow + ch * GC, GC)],
                         ea_v.at[sl], sems_e[sl])
        pltpu.async_copy(xa_hbm.at[src_v.at[pl.ds(ch * GC, GC)]],
                         rows_v.at[sl], sems_g[sl])

    def wait_slot(sl):
        pltpu.make_async_copy(ea_hbm.at[pl.ds(0, GC)], ea_v.at[sl],
                              sems_e[sl]).wait()
        pltpu.make_async_copy(xa_hbm.at[pl.ds(0, GC)], rows_v.at[sl],
                              sems_g[sl]).wait()

    def work(ch, sl):
        def grp(g, _):
            o = pl.multiple_of(g * L, L)
            d16 = dst_v[ch, pl.ds(o, L)]
            adv = plsc.load_gather(ad_v, [d16])
            for lane in range(L):
                e = o + lane
                tv = jnp.zeros((L,), jnp.float32)
                hjs = []
                for j in range(H // L):
                    sj = pl.ds(j * L, L)
                    v = (rows_v[sl, e, sj]
                         + ea_v[sl, e, pl.ds(pl.multiple_of(
                               ecol + j * L, L), L)])
                    hj = jnp.maximum(v, 0.01 * v)
                    hjs.append(hj)
                    tv = tv + hj * attl_v[sj]
                t = jnp.sum(tv) + adv[lane]
                av = jnp.full((L,), t, jnp.float32)
                exv = jnp.exp(jnp.maximum(av, 0.01 * av))
                for j in range(H // L):
                    msg_v[sl, e, pl.ds(j * L, L)] = hjs[j] * exv
                msg_v[sl, e, pl.ds(H, L)] = jnp.where(iota == 0, exv, 0.0)
            return 0

        lax.fori_loop(0, GC // L, grp, 0)
        pltpu.async_copy(msg_v.at[sl], acc_sh.at[dst_v.at[ch]], sems_s[sl],
                         add=True)

    def wait_scat(sl):
        pltpu.make_async_copy(msg_v.at[sl], acc_sh.at[dst_v.at[0]],
                              sems_s[sl]).wait()

    for c0 in range(NBUF - 1):
        issue(c0, c0)

    def chunk_body(ch, _):
        for par in range(NBUF):
            @pl.when(lax.rem(ch, NBUF) == par)
            def _():
                @pl.when(ch + NBUF - 1 < NCH)
                def _():
                    issue(ch + NBUF - 1, (par + NBUF - 1) % NBUF)
                wait_slot(par)

                @pl.when(ch >= NBUF)
                def _():
                    wait_scat(par)
                work(ch, par)
        return 0

    lax.fori_loop(0, NCH, chunk_body, 0)
    for sl in range(NBUF):
        wait_scat(sl)
    plsc.subcore_barrier()
    pltpu.sync_copy(acc_sh.at[pl.ds(sid * RPT, RPT)], out_hbm.at[cid, sid])


@functools.partial(
    pl.kernel,
    out_type=jax.ShapeDtypeStruct((NC, NS, RPT, AW), jnp.float32),
    mesh=_mesh,
    compiler_params=pltpu.CompilerParams(needs_layout_passes=False,
                                         use_tc_tiling_on_sc=False),
    scratch_types=[
        pltpu.VMEM((EPT,), jnp.int32),
        pltpu.VMEM((NCHP, GC), jnp.int32),
        pltpu.VMEM((NBUF, GC, H), jnp.float32),
        pltpu.VMEM((NBUF, GC, AW), jnp.float32),
        pltpu.VMEM((N,), jnp.float32),
        pltpu.VMEM((N,), jnp.float32),
        pltpu.VMEM_SHARED((N, AW), jnp.float32),
    ] + [pltpu.SemaphoreType.DMA] * (2 * NBUF),
)
def _att_edge_sc(x_hbm, src_hbm, dst2_hbm, as_hbm, ad_hbm, out_hbm,
                 src_v, dst_v, rows_v, msg_v, as_v, ad_v, acc_sh, *sems):
    cid = lax.axis_index("c")
    sid = lax.axis_index("s")
    wid = sid * NC + cid
    base = wid * EPT

    _zero_msg(msg_v.at[0], GC, AW)
    _zero_acc_stripe(msg_v.at[0], acc_sh, sid)

    pltpu.sync_copy(src_hbm.at[pl.ds(base, EPT)], src_v)
    pltpu.sync_copy(dst2_hbm.at[wid], dst_v)
    pltpu.sync_copy(as_hbm, as_v)
    pltpu.sync_copy(ad_hbm, ad_v)
    plsc.subcore_barrier()

    iota = lax.iota(jnp.int32, L)
    sems_g = sems[:NBUF]
    sems_s = sems[NBUF:]

    def issue(ch, sl):
        pltpu.async_copy(x_hbm.at[src_v.at[pl.ds(ch * GC, GC)]],
                         rows_v.at[sl], sems_g[sl])

    def wait_slot(sl):
        pltpu.make_async_copy(x_hbm.at[pl.ds(0, GC)], rows_v.at[sl],
                              sems_g[sl]).wait()

    def work(ch, sl):
        def grp(g, _):
            o = pl.multiple_of(g * L, L)
            s16 = src_v[pl.ds(pl.multiple_of(ch * GC + g * L, L), L)]
            d16 = dst_v[ch, pl.ds(o, L)]
            a = plsc.load_gather(as_v, [s16]) + plsc.load_gather(ad_v, [d16])
            exv16 = jnp.exp(jnp.maximum(a, 0.01 * a))
            for lane in range(L):
                e = o + lane
                exv = jnp.full((L,), exv16[lane], jnp.float32)
                for j in range(H // L):
                    sj = pl.ds(j * L, L)
                    msg_v[sl, e, sj] = rows_v[sl, e, sj] * exv
                msg_v[sl, e, pl.ds(H, L)] = jnp.where(iota == 0, exv, 0.0)
            return 0

        lax.fori_loop(0, GC // L, grp, 0)
        pltpu.async_copy(msg_v.at[sl], acc_sh.at[dst_v.at[ch]], sems_s[sl],
                         add=True)

    def wait_scat(sl):
        pltpu.make_async_copy(msg_v.at[sl], acc_sh.at[dst_v.at[0]],
                              sems_s[sl]).wait()

    for c0 in range(NBUF - 1):
        issue(c0, c0)

    def chunk_body(ch, _):
        for par in range(NBUF):
            @pl.when(lax.rem(ch, NBUF) == par)
            def _():
                @pl.when(ch + NBUF - 1 < NCH)
                def _():
                    issue(ch + NBUF - 1, (par + NBUF - 1) % NBUF)
                wait_slot(par)

                @pl.when(ch >= NBUF)
                def _():
                    wait_scat(par)
                work(ch, par)
        return 0

    lax.fori_loop(0, NCH, chunk_body, 0)
    for sl in range(NBUF):
        wait_scat(sl)
    plsc.subcore_barrier()
    pltpu.sync_copy(acc_sh.at[pl.ds(sid * RPT, RPT)], out_hbm.at[cid, sid])


@functools.partial(
    pl.kernel,
    out_type=jax.ShapeDtypeStruct((NW, B + 1, AW), jnp.float32),
    mesh=_mesh,
    compiler_params=pltpu.CompilerParams(needs_layout_passes=False, use_tc_tiling_on_sc=False),
    scratch_types=[
        pltpu.VMEM((RP3, H), jnp.float32),
        pltpu.VMEM((RP3,), jnp.int32),
        pltpu.VMEM((RP3,), jnp.float32),
        pltpu.VMEM((B + 16,), jnp.float32),
        pltpu.VMEM((B + 1, AW), jnp.float32),
    ],
)
def _pool_att_sc(x_hbm, b_hbm, as_hbm, adt_hbm, out_hbm,
                 rows_v, b_v, as_v, adt_v, acc_v):
    cid = lax.axis_index("c")
    sid = lax.axis_index("s")
    wid = sid * NC + cid
    base = wid * RP3

    z = jnp.zeros((L,), jnp.float32)

    def zrow(i, _):
        for j in range(AW // L):
            acc_v[i, pl.ds(j * L, L)] = z
        return 0

    lax.fori_loop(0, B + 1, zrow, 0)

    pltpu.sync_copy(x_hbm.at[pl.ds(base, RP3)], rows_v)
    pltpu.sync_copy(b_hbm.at[pl.ds(base, RP3)], b_v)
    pltpu.sync_copy(as_hbm.at[pl.ds(base, RP3)], as_v)
    pltpu.sync_copy(adt_hbm, adt_v)

    iota = lax.iota(jnp.int32, L)

    def pg(g, _):
        o = pl.multiple_of(g * L, L)
        b16 = b_v[pl.ds(o, L)]
        a = as_v[pl.ds(o, L)] + plsc.load_gather(adt_v, [b16])
        exv16 = jnp.exp(jnp.maximum(a, 0.01 * a))
        for lane in range(L):
            e = o + lane
            de = b16[lane]
            exv = jnp.full((L,), exv16[lane], jnp.float32)
            for j in range(H // L):
                sl = pl.ds(j * L, L)
                acc_v[de, sl] = acc_v[de, sl] + rows_v[e, sl] * exv
            sl = pl.ds(H, L)
            acc_v[de, sl] = acc_v[de, sl] + jnp.where(iota == 0, exv, 0.0)
        return 0

    lax.fori_loop(0, RP3 // L, pg, 0)

    pltpu.sync_copy(acc_v, out_hbm.at[wid])


NB = 1000
NGRID = N // NB


def _full(spec_shape):
    nd = len(spec_shape)
    return pl.BlockSpec(spec_shape, lambda *_: (0,) * nd)


def _gru_block(xv, h, Wih, Whh, bih, bhh):
    gi = jnp.dot(xv, Wih.T, preferred_element_type=jnp.float32) + bih
    gh = jnp.dot(h, Whh.T, preferred_element_type=jnp.float32) + bhh
    r = jax.nn.sigmoid(gi[:, :H] + gh[:, :H])
    z = jax.nn.sigmoid(gi[:, H:2 * H] + gh[:, H:2 * H])
    n_ = jnp.tanh(gi[:, 2 * H:] + r * gh[:, 2 * H:])
    return (1.0 - z) * n_ + z * h


def _elu(v):
    return jnp.where(v > 0, v, jnp.exp(jnp.minimum(v, 0.0)) - 1.0)


def _tk_node_prep(x, W1, b1, W1a, attr):
    def body(x_r, W1_r, b1_r, W1a_r, attr_r, x1_r, xa_r, adg_r):
        x1 = _lk(jnp.dot(x_r[...], W1_r[...].T,
                         preferred_element_type=jnp.float32) + b1_r[...])
        x1_r[...] = x1
        xa_r[...] = jnp.dot(x1, W1a_r[...].T,
                            preferred_element_type=jnp.float32)
        adg_r[...] = jnp.sum(x1 * attr_r[...].T, axis=1, keepdims=True)

    return pl.pallas_call(
        body,
        grid=(NGRID,),
        in_specs=[pl.BlockSpec((NB, 128), lambda i: (i, 0)),
                  _full((H, 128)), _full((H,)), _full((H, H)),
                  _full((H, 1))],
        out_specs=[pl.BlockSpec((NB, H), lambda i: (i, 0)),
                   pl.BlockSpec((NB, H), lambda i: (i, 0)),
                   pl.BlockSpec((NB, 1), lambda i: (i, 0))],
        out_shape=[jax.ShapeDtypeStruct((N, H), jnp.float32),
                   jax.ShapeDtypeStruct((N, H), jnp.float32),
                   jax.ShapeDtypeStruct((N, 1), jnp.float32)],
    )(x, W1, b1, W1a, attr)


EB = 3200


def _tk_ea(edge_attr, W1b):
    dn = (((0,), (1,)), ((), ()))

    def body(lo_r, hi_r, w_r, o_r):
        lo = lax.dot_general(lo_r[...], w_r[...], dn,
                             preferred_element_type=jnp.float32)
        hi = lax.dot_general(hi_r[...], w_r[...], dn,
                             preferred_element_type=jnp.float32)
        o_r[...] = jnp.concatenate([lo, hi], axis=1)

    ea_t = edge_attr.T
    return pl.pallas_call(
        body,
        grid=(E // 2 // EB,),
        in_specs=[pl.BlockSpec((16, EB), lambda i: (0, i)),
                  pl.BlockSpec((16, EB), lambda i: (0, i + E // 2 // EB)),
                  _full((H, 16))],
        out_specs=pl.BlockSpec((EB, 2 * H), lambda i: (i, 0)),
        out_shape=jax.ShapeDtypeStruct((E // 2, 2 * H), jnp.float32),
    )(ea_t, ea_t, W1b)


def _tk_conv_post(acc, xprev, Wc, bc, Wih, Whh, bih, bhh, Wn, asv, adv):
    def body(a0_r, a1_r, xp_r, Wc_r, bc_r, Wih_r, Whh_r, bih_r, bhh_r,
             Wn_r, as0_r, ad0_r, xn_r, as_r, ad_r):
        a = a0_r[0] + a1_r[0]
        u = a[:, :H] / (a[:, H:H + 1] + 1e-16)
        h = _elu(jnp.dot(u, Wc_r[...].T,
                         preferred_element_type=jnp.float32) + bc_r[...])
        xn = jnp.maximum(
            _gru_block(h, xp_r[...], Wih_r[...], Whh_r[...], bih_r[...],
                       bhh_r[...]), 0.0)
        xn_r[...] = xn
        ws = jnp.dot(Wn_r[...].T, as0_r[...],
                     preferred_element_type=jnp.float32)
        wd = jnp.dot(Wn_r[...].T, ad0_r[...],
                     preferred_element_type=jnp.float32)
        as_r[...] = jnp.sum(xn * ws.T, axis=1, keepdims=True)
        ad_r[...] = jnp.sum(xn * wd.T, axis=1, keepdims=True)

    return pl.pallas_call(
        body,
        grid=(NGRID,),
        in_specs=[pl.BlockSpec((1, NB, AW), lambda i: (0, i, 0)),
                  pl.BlockSpec((1, NB, AW), lambda i: (1, i, 0)),
                  pl.BlockSpec((NB, H), lambda i: (i, 0)),
                  _full((H, H)), _full((H,)),
                  _full((3 * H, H)), _full((3 * H, H)),
                  _full((3 * H,)), _full((3 * H,)),
                  _full((H, H)), _full((H, 1)), _full((H, 1))],
        out_specs=[pl.BlockSpec((NB, H), lambda i: (i, 0)),
                   pl.BlockSpec((NB, 1), lambda i: (i, 0)),
                   pl.BlockSpec((NB, 1), lambda i: (i, 0))],
        out_shape=[jax.ShapeDtypeStruct((N, H), jnp.float32),
                   jax.ShapeDtypeStruct((N, 1), jnp.float32),
                   jax.ShapeDtypeStruct((N, 1), jnp.float32)],
    )(acc, acc, xprev, Wc, bc, Wih, Whh, bih, bhh, Wn, asv, adv)


def _tk_pool_post(p32, Wmc, attd):
    def body(p_r, W_r, ad_r, og_r, adg_r):
        seg = jnp.sum(p_r[...], axis=0)[:B, :H]
        og = jnp.maximum(seg, 0.0)
        og_r[...] = og
        wv = jnp.dot(W_r[...].T, ad_r[...],
                     preferred_element_type=jnp.float32)
        adg_r[...] = jnp.sum(og * wv.T, axis=1, keepdims=True)

    return pl.pallas_call(
        body,
        in_specs=[_full((NW, B + 1, AW)), _full((H, H)), _full((H, 1))],
        out_specs=[_full((B, H)), _full((B, 1))],
        out_shape=[jax.ShapeDtypeStruct((B, H), jnp.float32),
                   jax.ShapeDtypeStruct((B, 1), jnp.float32)],
    )(p32, Wmc, attd)


def _tk_mc_post(m32, og, Wmc, bmc, Wih, Whh, bih, bhh, attd):
    def body(m_r, og_r, W_r, b_r, Wih_r, Whh_r, bih_r, bhh_r, ad_r,
             on_r, adg_r):
        a = jnp.sum(m_r[...], axis=0)[:B]
        u = a[:, :H] / (a[:, H:H + 1] + 1e-16)
        h = _elu(jnp.dot(u, W_r[...].T,
                         preferred_element_type=jnp.float32) + b_r[...])
        on = jnp.maximum(
            _gru_block(h, og_r[...], Wih_r[...], Whh_r[...], bih_r[...],
                       bhh_r[...]), 0.0)
        on_r[...] = on
        wv = jnp.dot(W_r[...].T, ad_r[...],
                     preferred_element_type=jnp.float32)
        adg_r[...] = jnp.sum(on * wv.T, axis=1, keepdims=True)

    return pl.pallas_call(
        body,
        in_specs=[_full((NW, B + 1, AW)), _full((B, H)),
                  _full((H, H)), _full((H,)),
                  _full((3 * H, H)), _full((3 * H, H)),
                  _full((3 * H,)), _full((3 * H,)), _full((H, 1))],
        out_specs=[_full((B, H)), _full((B, 1))],
        out_shape=[jax.ShapeDtypeStruct((B, H), jnp.float32),
                   jax.ShapeDtypeStruct((B, 1), jnp.float32)],
    )(m32, og, Wmc, bmc, Wih, Whh, bih, bhh, attd)


GB = 64
GL = 3072


def _tk_gene(gene, gc_W, gc_b):
    def body(g_r, w_r, b_r, o_r):
        m_idx = lax.broadcasted_iota(jnp.int32, (GL, H), 0)
        i_idx = lax.broadcasted_iota(jnp.int32, (GL, H), 1)
        mask = (m_idx // (GL // H) == i_idx).astype(jnp.float32) / 16.0
        rem = m_idx % 3
        acc = jnp.zeros((GB, H), jnp.float32)
        w = w_r[...]
        for c in range(4):
            vals = jnp.where(rem == 0, w[c, 0],
                             jnp.where(rem == 1, w[c, 1], w[c, 2]))
            acc = acc + jnp.dot(g_r[0, :, c, :], vals * mask,
                                preferred_element_type=jnp.float32)
        o_r[...] = acc + b_r[0]

    return pl.pallas_call(
        body,
        grid=(B // GB,),
        in_specs=[pl.BlockSpec((1, GB, 4, GL), lambda i: (0, i, 0, 0)),
                  _full((4, 3)), _full((1,))],
        out_specs=pl.BlockSpec((GB, H), lambda i: (i, 0)),
        out_shape=jax.ShapeDtypeStruct((B, H), jnp.float32),
    )(gene[None], gc_W, gc_b)


def _tk_final(og, gp, taxonomy, duration, W_dur, b_dur, W4, b4, W5, b5):
    def body(og_r, gp_r, tx_r, du_r, Wd_r, bd_r, W4_r, b4_r, W5_r, b5_r,
             o_r):
        dur = jnp.maximum(
            jnp.dot(du_r[...], Wd_r[...].T,
                    preferred_element_type=jnp.float32) + bd_r[...], 0.0)
        cat = jnp.concatenate([og_r[...], gp_r[...], tx_r[...], dur], 1)
        c4 = jnp.dot(cat, W4_r[...].T,
                     preferred_element_type=jnp.float32) + b4_r[...]
        o_r[...] = jnp.sum(c4 * W5_r[...], axis=1, keepdims=True) + b5_r[0]

    return pl.pallas_call(
        body,
        in_specs=[_full((B, H)), _full((B, H)), _full((B, H)),
                  _full((B, DUR)), _full((H, DUR)), _full((H,)),
                  _full((H, 4 * H)), _full((H,)),
                  _full((1, H)), _full((1,))],
        out_specs=_full((B, 1)),
        out_shape=jax.ShapeDtypeStruct((B, 1), jnp.float32),
    )(og, gp, taxonomy, duration, W_dur, b_dur, W4, b4, W5, b5)


def _gru(xv, h, Wih, Whh, bih, bhh):
    gi = xv @ Wih.T + bih
    gh = h @ Whh.T + bhh
    ir, iz, inn = jnp.split(gi, 3, axis=1)
    hr, hz, hn = jnp.split(gh, 3, axis=1)
    r = jax.nn.sigmoid(ir + hr)
    zz = jax.nn.sigmoid(iz + hz)
    n_ = jnp.tanh(inn + r * hn)
    return (1.0 - zz) * n_ + zz * h


def kernel(x, edge_index, edge_attr, batch, gene, taxonomy, duration,
           W1, b1, ge_W1, ge_W2, ge_att_l, ge_att_r, ge_bias,
           gru_Wih, gru_Whh, gru_bih, gru_bhh,
           ac_W, ac_att_src, ac_att_dst, ac_bias,
           ag_Wih, ag_Whh, ag_bih, ag_bhh,
           mc_W, mc_att_src, mc_att_dst, mc_bias,
           mg_Wih, mg_Whh, mg_bih, mg_bhh,
           gc_W, gc_b, W_dur, b_dur, W4, b4, W5, b5):
    f32 = jnp.float32
    src = edge_index[0]
    dst = edge_index[1]
    dst2 = jnp.pad(dst.reshape(NW, NCH, GC), ((0, 0), (0, NCHP - NCH), (0, 0)))

    W1a = ge_W1[:, :H]
    W1b = ge_W1[:, H:]
    x1, xa, adg = _tk_node_prep(x, W1, b1, W1a, ge_att_r.reshape(H, 1))
    ea = _tk_ea(edge_attr, W1b)

    acc = _gate_edge_sc(xa, ea, src, dst2, adg.reshape(N), ge_att_l)
    acc = acc.reshape(NC, N, AW)
    x2, as2, ad2 = _tk_conv_post(acc, x1, ge_W2, ge_bias,
                                 gru_Wih, gru_Whh, gru_bih, gru_bhh,
                                 ac_W, ac_att_src.reshape(H, 1),
                                 ac_att_dst.reshape(H, 1))

    acc2 = _att_edge_sc(x2, src, dst2, as2.reshape(N),
                        ad2.reshape(N)).reshape(NC, N, AW)
    x3, as3, _ = _tk_conv_post(acc2, x2, ac_W, ac_bias,
                               ag_Wih, ag_Whh, ag_bih, ag_bhh,
                               mc_W, mc_att_src.reshape(H, 1),
                               mc_att_src.reshape(H, 1))

    x3p = jnp.concatenate([x3, jnp.zeros((NP - N, H), f32)], 0)
    bp = jnp.concatenate([batch, jnp.full((NP - N,), B, jnp.int32)], 0)
    as3p = jnp.concatenate([as3.reshape(N), jnp.zeros((NP - N,), f32)], 0)
    zs = jnp.zeros((NP,), f32)
    zt = jnp.zeros((B + 16,), f32)

    p32 = _pool_att_sc(x3p, bp, zs, zt)
    out_g, adg1 = _tk_pool_post(p32, mc_W, mc_att_dst.reshape(H, 1))

    adgp = adg1
    for _ in range(2):
        m32 = _pool_att_sc(x3p, bp, as3p,
                           jnp.pad(adgp.reshape(B), (0, 16)))
        out_g, adgp = _tk_mc_post(m32, out_g, mc_W, mc_bias,
                                  mg_Wih, mg_Whh, mg_bih, mg_bhh,
                                  mc_att_dst.reshape(H, 1))

    gp = _tk_gene(gene, gc_W, gc_b)
    return _tk_final(out_g, gp, taxonomy, duration, W_dur, b_dur,
                     W4, b4, W5, b5)
